# Initial kernel scaffold; baseline (speedup 1.0000x reference)
import jax
import jax.numpy as jnp
from jax.experimental import pallas as pl


def kernel(h, x, edges, params):
    raise NotImplementedError("write your pallas kernel here")


# speedup vs baseline: 3.8589x; 3.8589x over previous
import jax
import jax.numpy as jnp
from jax import lax
from jax.experimental import pallas as pl
from jax.experimental.pallas import tpu as pltpu
from jax.experimental.pallas import tpu_sc as plsc

N = 10000
E = 320000
D = 128
L = 4
CP = 16
NP = 10240
NC = 2
NS = 16
NW = NC * NS
CH = 128
NCHUNK = E // CH
FULL_W = NCHUNK // NW
EXTRA_W = NCHUNK - FULL_W * NW
NCHUNK_SC = NCHUNK // NC
FULL_T = NCHUNK_SC // NS
EXTRA_T = NCHUNK_SC - FULL_T * NS


def _sc_gather_body(t1, t2, row, col, g1, g2, idx_r, idx_c, buf1, buf2, s1, s2):
    wid = lax.axis_index("s") * NC + lax.axis_index("c")

    def do_chunk(base):
        pltpu.sync_copy(row.at[pl.ds(base, CH)], idx_r)
        pltpu.sync_copy(col.at[pl.ds(base, CH)], idx_c)
        cp1 = pltpu.async_copy(t1.at[idx_r], buf1, s1)
        cp2 = pltpu.async_copy(t2.at[idx_c], buf2, s2)
        cp1.wait()
        cp2.wait()
        pltpu.sync_copy(buf1, g1.at[pl.ds(base, CH)])
        pltpu.sync_copy(buf2, g2.at[pl.ds(base, CH)])

    def body(k, carry):
        do_chunk((wid + k * NW) * CH)
        return carry

    lax.fori_loop(0, FULL_W, body, 0)

    @pl.when(wid < EXTRA_W)
    def _():
        do_chunk((wid + FULL_W * NW) * CH)


def _sc_scatter_body(val, row, zeros, out0, out1, idx_v, buf, acc):
    cid = lax.axis_index("c")
    sid = lax.axis_index("s")
    rs = NP // NS

    pltpu.sync_copy(zeros.at[pl.ds(sid * rs, rs)], acc.at[pl.ds(sid * rs, rs)])
    plsc.subcore_barrier()

    def do_chunk(base):
        pltpu.sync_copy(row.at[pl.ds(base, CH)], idx_v)
        pltpu.sync_copy(val.at[pl.ds(base, CH)], buf)
        pltpu.sync_copy(buf, acc.at[idx_v], add=True)

    def body(k, carry):
        do_chunk(cid * (E // NC) + (sid + k * NS) * CH)
        return carry

    lax.fori_loop(0, FULL_T, body, 0)

    @pl.when(sid < EXTRA_T)
    def _():
        do_chunk(cid * (E // NC) + (sid + FULL_T * NS) * CH)

    plsc.subcore_barrier()

    @pl.when(cid == 0)
    def _():
        pltpu.sync_copy(acc.at[pl.ds(sid * rs, rs)], out0.at[pl.ds(sid * rs, rs)])

    @pl.when(cid == 1)
    def _():
        pltpu.sync_copy(acc.at[pl.ds(sid * rs, rs)], out1.at[pl.ds(sid * rs, rs)])


_sc_cache = {}


def _sc_gather(t1, t2, row, col):
    if "gather" not in _sc_cache:
        mesh = plsc.VectorSubcoreMesh(core_axis_name="c", subcore_axis_name="s")
        _sc_cache["gather"] = pl.kernel(
            _sc_gather_body,
            mesh=mesh,
            out_type=(
                jax.ShapeDtypeStruct((E, D), jnp.int32),
                jax.ShapeDtypeStruct((E, D), jnp.int32),
            ),
            scratch_types=[
                pltpu.VMEM((CH,), jnp.int32),
                pltpu.VMEM((CH,), jnp.int32),
                pltpu.VMEM((CH, D), jnp.int32),
                pltpu.VMEM((CH, D), jnp.int32),
                pltpu.SemaphoreType.DMA,
                pltpu.SemaphoreType.DMA,
            ],
        )
    return _sc_cache["gather"](t1, t2, row, col)


def _sc_scatter(val, row, zeros):
    if "scatter" not in _sc_cache:
        mesh = plsc.VectorSubcoreMesh(core_axis_name="c", subcore_axis_name="s")
        _sc_cache["scatter"] = pl.kernel(
            _sc_scatter_body,
            mesh=mesh,
            out_type=(
                jax.ShapeDtypeStruct((NP, D), jnp.float32),
                jax.ShapeDtypeStruct((NP, D), jnp.float32),
            ),
            scratch_types=[
                pltpu.VMEM((CH,), jnp.int32),
                pltpu.VMEM((CH, D), jnp.float32),
                pltpu.VMEM_SHARED((NP, D), jnp.float32),
            ],
        )
    return _sc_cache["scatter"](val, row, zeros)


def _pack_words(payload, coord3):
    pay = lax.convert_element_type(
        lax.bitcast_convert_type(payload.astype(jnp.bfloat16), jnp.uint16),
        jnp.uint32)
    cbits = lax.bitcast_convert_type(coord3, jnp.uint32)
    hi = cbits >> 16
    lo = cbits & 0xFFFF
    b = payload.shape[0]
    z13 = jnp.zeros((b, 13), jnp.uint32)
    z109 = jnp.zeros((b, 109), jnp.uint32)
    cplane = jnp.concatenate([hi, z13, lo, z109], axis=1)
    return lax.bitcast_convert_type(pay | (cplane << 16), jnp.int32)


def _unpack_words(words):
    w = lax.bitcast_convert_type(words, jnp.uint32)
    pay = lax.bitcast_convert_type(
        lax.convert_element_type(w & 0xFFFF, jnp.uint16),
        jnp.bfloat16).astype(jnp.float32)
    cplane = w >> 16
    coord = lax.bitcast_convert_type(
        (cplane[:, 0:3] << 16) | cplane[:, 16:19], jnp.float32)
    return pay, coord


BE = 3200


def _sigmoid(x):
    return 1.0 / (1.0 + jnp.exp(-x))


def _silu(x):
    return x * _sigmoid(x)


def _mmt(x, w):
    return lax.dot_general(x, w, (((1,), (1,)), ((), ())),
                           preferred_element_type=jnp.float32)


def _edge_body(g1_ref, g2_ref, wr_ref, b1_ref, w2_ref, b2_ref,
               cw1_ref, cb1_ref, cw2_ref, vh_ref, vc_ref):
    h1, c1 = _unpack_words(g1_ref[...])
    h2, c2 = _unpack_words(g2_ref[...])
    hs = h1 + h2
    cd = c1 - c2
    radial = jnp.sum(cd * cd, axis=1, keepdims=True)
    t = _silu(hs + radial * wr_ref[...] + b1_ref[...])
    ef = _silu(_mmt(t, w2_ref[...]) + b2_ref[...])
    cm = _silu(_mmt(ef, cw1_ref[...]) + cb1_ref[...])
    cms = _mmt(cm, cw2_ref[...])
    vh_ref[...] = ef
    ones = jnp.ones((BE, 1), jnp.float32)
    z124 = jnp.zeros((BE, D - 4), jnp.float32)
    vc_ref[...] = jnp.concatenate([cd * cms, ones, z124], axis=1)


def _edge_stage(g1, g2, wr, b1, w2, b2, cw1, cb1, cw2):
    wspec = lambda shape: pl.BlockSpec(shape, lambda i: (0, 0))
    return pl.pallas_call(
        _edge_body,
        grid=(E // BE,),
        in_specs=[
            pl.BlockSpec((BE, D), lambda i: (i, 0)),
            pl.BlockSpec((BE, D), lambda i: (i, 0)),
            wspec((1, D)), wspec((1, D)), wspec((D, D)), wspec((1, D)),
            wspec((D, D)), wspec((1, D)), wspec((1, D)),
        ],
        out_specs=[
            pl.BlockSpec((BE, D), lambda i: (i, 0)),
            pl.BlockSpec((BE, D), lambda i: (i, 0)),
        ],
        out_shape=[
            jax.ShapeDtypeStruct((E, D), jnp.float32),
            jax.ShapeDtypeStruct((E, D), jnp.float32),
        ],
    )(g1, g2, wr, b1, w2, b2, cw1, cb1, cw2)


BN = 1024


def _pack_tables(h_new, coord_new, wa, wb):
    t1 = _pack_words(_mmt(h_new, wa), coord_new)
    t2 = _pack_words(_mmt(h_new, wb), coord_new)
    return t1, t2


def _node_common(p0, p1, q0, q1, cp, h, nw1a, nw1b, nb1, nw2, nb2):
    agg = p0 + p1
    sc = q0 + q1
    cnt = jnp.maximum(sc[:, 3:4], 1.0)
    coord_new = cp[:, :3] + sc[:, :3] / cnt
    m = _silu(_mmt(h, nw1a) + _mmt(agg, nw1b) + nb1)
    h_new = h + _mmt(m, nw2) + nb2
    return h_new, coord_new


def _node_body(p0_ref, p1_ref, q0_ref, q1_ref, h_ref, cp_ref,
               nw1a_ref, nw1b_ref, nb1_ref, nw2_ref, nb2_ref,
               wa_ref, wb_ref, h_out, cp_out, t1_out, t2_out):
    h_new, coord_new = _node_common(
        p0_ref[...], p1_ref[...], q0_ref[...], q1_ref[...], cp_ref[...],
        h_ref[...], nw1a_ref[...], nw1b_ref[...], nb1_ref[...],
        nw2_ref[...], nb2_ref[...])
    h_out[...] = h_new
    cp_out[...] = jnp.concatenate(
        [coord_new, jnp.zeros((BN, CP - 3), jnp.float32)], axis=1)
    t1, t2 = _pack_tables(h_new, coord_new, wa_ref[...], wb_ref[...])
    t1_out[...] = t1
    t2_out[...] = t2


def _node_stage(p0, p1, q0, q1, h, cpad, nw1a, nw1b, nb1, nw2, nb2, wa, wb):
    wspec = lambda shape: pl.BlockSpec(shape, lambda i: (0, 0))
    bspec = lambda w: pl.BlockSpec((BN, w), lambda i: (i, 0))
    return pl.pallas_call(
        _node_body,
        grid=(NP // BN,),
        in_specs=[
            bspec(D), bspec(D), bspec(D), bspec(D), bspec(D), bspec(CP),
            wspec((D, D)), wspec((D, D)), wspec((1, D)),
            wspec((D, D)), wspec((1, D)), wspec((D, D)), wspec((D, D)),
        ],
        out_specs=[
            bspec(D), bspec(CP), bspec(D), bspec(D),
        ],
        out_shape=[
            jax.ShapeDtypeStruct((NP, D), jnp.float32),
            jax.ShapeDtypeStruct((NP, CP), jnp.float32),
            jax.ShapeDtypeStruct((NP, D), jnp.int32),
            jax.ShapeDtypeStruct((NP, D), jnp.int32),
        ],
    )(p0, p1, q0, q1, h, cpad, nw1a, nw1b, nb1, nw2, nb2, wa, wb)


def _final_body(p0_ref, p1_ref, q0_ref, q1_ref, h_ref, cp_ref,
                nw1a_ref, nw1b_ref, nb1_ref, nw2_ref, nb2_ref,
                ow_ref, ob_ref, h_out, c_out):
    h_new, coord_new = _node_common(
        p0_ref[...], p1_ref[...], q0_ref[...], q1_ref[...], cp_ref[...],
        h_ref[...], nw1a_ref[...], nw1b_ref[...], nb1_ref[...],
        nw2_ref[...], nb2_ref[...])
    h_out[...] = _mmt(h_new, ow_ref[...]) + ob_ref[...]
    c_out[...] = jnp.concatenate(
        [coord_new, jnp.zeros((BN, CP - 3), jnp.float32)], axis=1)


def _final_stage(p0, p1, q0, q1, h, cpad, nw1a, nw1b, nb1, nw2, nb2, ow, ob):
    wspec = lambda shape: pl.BlockSpec(shape, lambda i: (0, 0))
    bspec = lambda w: pl.BlockSpec((BN, w), lambda i: (i, 0))
    return pl.pallas_call(
        _final_body,
        grid=(NP // BN,),
        in_specs=[
            bspec(D), bspec(D), bspec(D), bspec(D), bspec(D), bspec(CP),
            wspec((D, D)), wspec((D, D)), wspec((1, D)),
            wspec((D, D)), wspec((1, D)), wspec((D, D)), wspec((1, D)),
        ],
        out_specs=[bspec(D), bspec(CP)],
        out_shape=[
            jax.ShapeDtypeStruct((NP, D), jnp.float32),
            jax.ShapeDtypeStruct((NP, CP), jnp.float32),
        ],
    )(p0, p1, q0, q1, h, cpad, nw1a, nw1b, nb1, nw2, nb2, ow, ob)


def _prep_body(h_ref, cp_ref, wa_ref, wb_ref, t1_out, t2_out):
    t1, t2 = _pack_tables(h_ref[...], cp_ref[:, :3], wa_ref[...], wb_ref[...])
    t1_out[...] = t1
    t2_out[...] = t2


def _prep_stage(h, cpad, wa, wb):
    wspec = lambda shape: pl.BlockSpec(shape, lambda i: (0, 0))
    return pl.pallas_call(
        _prep_body,
        grid=(NP // BN,),
        in_specs=[
            pl.BlockSpec((BN, D), lambda i: (i, 0)),
            pl.BlockSpec((BN, CP), lambda i: (i, 0)),
            wspec((D, D)), wspec((D, D)),
        ],
        out_specs=[
            pl.BlockSpec((BN, D), lambda i: (i, 0)),
            pl.BlockSpec((BN, D), lambda i: (i, 0)),
        ],
        out_shape=[
            jax.ShapeDtypeStruct((NP, D), jnp.int32),
            jax.ShapeDtypeStruct((NP, D), jnp.int32),
        ],
    )(h, cpad, wa, wb)


def kernel(h, x, edges, params):
    row = edges[0]
    col = edges[1]
    h_pad = jnp.pad(h, ((0, NP - N), (0, 0)))
    cpad = jnp.pad(x, ((0, NP - N), (0, CP - 3)))
    zeros = jnp.zeros((NP, D), jnp.float32)

    def layer_w(i):
        ew1 = params[f"ew1_{i}"]
        wa = ew1[:, :D]
        wb = ew1[:, D:2 * D]
        wr = ew1[:, 2 * D:].reshape(1, D)
        b1 = params[f"eb1_{i}"].reshape(1, D)
        w2 = params[f"ew2_{i}"]
        b2 = params[f"eb2_{i}"].reshape(1, D)
        nw1 = params[f"nw1_{i}"]
        nw1a = nw1[:, :D]
        nw1b = nw1[:, D:]
        nb1 = params[f"nb1_{i}"].reshape(1, D)
        nw2 = params[f"nw2_{i}"]
        nb2 = params[f"nb2_{i}"].reshape(1, D)
        cw1 = params[f"cw1_{i}"]
        cb1 = params[f"cb1_{i}"].reshape(1, D)
        cw2 = params[f"cw2_{i}"]
        return wa, wb, wr, b1, w2, b2, nw1a, nw1b, nb1, nw2, nb2, cw1, cb1, cw2

    wa0, wb0 = layer_w(0)[:2]
    t1, t2 = _prep_stage(h_pad, cpad, wa0, wb0)

    for i in range(L):
        wa, wb, wr, b1, w2, b2, nw1a, nw1b, nb1, nw2, nb2, cw1, cb1, cw2 = layer_w(i)
        g1, g2 = _sc_gather(t1, t2, row, col)
        val_h, val_c = _edge_stage(g1, g2, wr, b1, w2, b2, cw1, cb1, cw2)
        p0, p1 = _sc_scatter(val_h, row, zeros)
        q0, q1 = _sc_scatter(val_c, row, zeros)
        if i < L - 1:
            wa_n, wb_n = layer_w(i + 1)[:2]
            h_pad, cpad, t1, t2 = _node_stage(
                p0, p1, q0, q1, h_pad, cpad, nw1a, nw1b, nb1, nw2, nb2,
                wa_n, wb_n)
        else:
            h_fin, c_fin = _final_stage(
                p0, p1, q0, q1, h_pad, cpad, nw1a, nw1b, nb1, nw2, nb2,
                params["out_w"], params["out_b"].reshape(1, D))

    return (h_fin[:N], c_fin[:N, :3])


# speedup vs baseline: 4.2525x; 1.1020x over previous
import jax
import jax.numpy as jnp
from jax import lax
from jax.experimental import pallas as pl
from jax.experimental.pallas import tpu as pltpu
from jax.experimental.pallas import tpu_sc as plsc

N = 10000
E = 320000
D = 128
L = 4
CP = 16
NP = 10240
NC = 2
NS = 16
NW = NC * NS
CH = 128
NCHUNK = E // CH
FULL_W = NCHUNK // NW
EXTRA_W = NCHUNK - FULL_W * NW
NCHUNK_SC = NCHUNK // NC
FULL_T = NCHUNK_SC // NS
EXTRA_T = NCHUNK_SC - FULL_T * NS


def _sc_gather_body(t1, t2, row, col, g1, g2, idx_r, idx_c, buf1, buf2, s1, s2):
    wid = lax.axis_index("s") * NC + lax.axis_index("c")

    def do_chunk(base):
        pltpu.sync_copy(row.at[pl.ds(base, CH)], idx_r)
        pltpu.sync_copy(col.at[pl.ds(base, CH)], idx_c)
        cp1 = pltpu.async_copy(t1.at[idx_r], buf1, s1)
        cp2 = pltpu.async_copy(t2.at[idx_c], buf2, s2)
        cp1.wait()
        cp2.wait()
        pltpu.sync_copy(buf1, g1.at[pl.ds(base, CH)])
        pltpu.sync_copy(buf2, g2.at[pl.ds(base, CH)])

    def body(k, carry):
        do_chunk((wid + k * NW) * CH)
        return carry

    lax.fori_loop(0, FULL_W, body, 0)

    @pl.when(wid < EXTRA_W)
    def _():
        do_chunk((wid + FULL_W * NW) * CH)


def _sc_scatter_body(val, vct, row, zeros, zeros_c, out0, out1, cout0, cout1,
                     idx_v, eidx_v, buf, tbuf, acc, accc):
    cid = lax.axis_index("c")
    sid = lax.axis_index("s")
    rs = NP // NS
    cs = (4 * NP) // NS

    pltpu.sync_copy(zeros.at[pl.ds(sid * rs, rs)], acc.at[pl.ds(sid * rs, rs)])
    pltpu.sync_copy(zeros_c.at[pl.ds(sid * cs, cs)],
                    accc.at[pl.ds(sid * cs, cs)])
    plsc.subcore_barrier()

    def do_chunk(ci):
        base = ci * CH
        pltpu.sync_copy(row.at[pl.ds(base, CH)], idx_v)
        pltpu.sync_copy(val.at[pl.ds(base, CH)], buf)
        pltpu.sync_copy(buf, acc.at[idx_v], add=True)
        pltpu.sync_copy(vct.at[ci], tbuf)
        for c in range(4):
            for j in range(8):
                sl = pl.ds(j * 16, 16)
                eidx_v[sl] = idx_v[sl] + c * NP
            pltpu.sync_copy(tbuf.at[c], accc.at[eidx_v], add=True)

    def body(k, carry):
        do_chunk(cid * NCHUNK_SC + sid + k * NS)
        return carry

    lax.fori_loop(0, FULL_T, body, 0)

    @pl.when(sid < EXTRA_T)
    def _():
        do_chunk(cid * NCHUNK_SC + sid + FULL_T * NS)

    plsc.subcore_barrier()

    @pl.when(cid == 0)
    def _():
        pltpu.sync_copy(acc.at[pl.ds(sid * rs, rs)], out0.at[pl.ds(sid * rs, rs)])
        pltpu.sync_copy(accc.at[pl.ds(sid * cs, cs)],
                        cout0.at[pl.ds(sid * cs, cs)])

    @pl.when(cid == 1)
    def _():
        pltpu.sync_copy(acc.at[pl.ds(sid * rs, rs)], out1.at[pl.ds(sid * rs, rs)])
        pltpu.sync_copy(accc.at[pl.ds(sid * cs, cs)],
                        cout1.at[pl.ds(sid * cs, cs)])


_sc_cache = {}


def _sc_gather(t1, t2, row, col):
    if "gather" not in _sc_cache:
        mesh = plsc.VectorSubcoreMesh(core_axis_name="c", subcore_axis_name="s")
        _sc_cache["gather"] = pl.kernel(
            _sc_gather_body,
            mesh=mesh,
            out_type=(
                jax.ShapeDtypeStruct((E, D), jnp.int32),
                jax.ShapeDtypeStruct((E, D), jnp.int32),
            ),
            scratch_types=[
                pltpu.VMEM((CH,), jnp.int32),
                pltpu.VMEM((CH,), jnp.int32),
                pltpu.VMEM((CH, D), jnp.int32),
                pltpu.VMEM((CH, D), jnp.int32),
                pltpu.SemaphoreType.DMA,
                pltpu.SemaphoreType.DMA,
            ],
        )
    return _sc_cache["gather"](t1, t2, row, col)


def _sc_scatter(val, vct, row, zeros, zeros_c):
    if "scatter" not in _sc_cache:
        mesh = plsc.VectorSubcoreMesh(core_axis_name="c", subcore_axis_name="s")
        _sc_cache["scatter"] = pl.kernel(
            _sc_scatter_body,
            mesh=mesh,
            out_type=(
                jax.ShapeDtypeStruct((NP, D), jnp.float32),
                jax.ShapeDtypeStruct((NP, D), jnp.float32),
                jax.ShapeDtypeStruct((4 * NP,), jnp.float32),
                jax.ShapeDtypeStruct((4 * NP,), jnp.float32),
            ),
            scratch_types=[
                pltpu.VMEM((CH,), jnp.int32),
                pltpu.VMEM((CH,), jnp.int32),
                pltpu.VMEM((CH, D), jnp.float32),
                pltpu.VMEM((8, CH), jnp.float32),
                pltpu.VMEM_SHARED((NP, D), jnp.float32),
                pltpu.VMEM_SHARED((4 * NP,), jnp.float32),
            ],
        )
    return _sc_cache["scatter"](val, vct, row, zeros, zeros_c)


def _pack_words(payload, coord3):
    pay = lax.convert_element_type(
        lax.bitcast_convert_type(payload.astype(jnp.bfloat16), jnp.uint16),
        jnp.uint32)
    cbits = lax.bitcast_convert_type(coord3, jnp.uint32)
    hi = cbits >> 16
    lo = cbits & 0xFFFF
    b = payload.shape[0]
    z13 = jnp.zeros((b, 13), jnp.uint32)
    z109 = jnp.zeros((b, 109), jnp.uint32)
    cplane = jnp.concatenate([hi, z13, lo, z109], axis=1)
    return lax.bitcast_convert_type(pay | (cplane << 16), jnp.int32)


def _unpack_words(words):
    w = lax.bitcast_convert_type(words, jnp.uint32)
    pay = lax.bitcast_convert_type(
        lax.convert_element_type(w & 0xFFFF, jnp.uint16),
        jnp.bfloat16).astype(jnp.float32)
    cplane = w >> 16
    coord = lax.bitcast_convert_type(
        (cplane[:, 0:3] << 16) | cplane[:, 16:19], jnp.float32)
    return pay, coord


BE = 3200


def _sigmoid(x):
    return 1.0 / (1.0 + jnp.exp(-x))


def _silu(x):
    return x * _sigmoid(x)


def _mmt(x, w):
    return lax.dot_general(x, w, (((1,), (1,)), ((), ())),
                           preferred_element_type=jnp.float32)


def _edge_body(g1_ref, g2_ref, wr_ref, b1_ref, w2_ref, b2_ref,
               cw1_ref, cb1_ref, cw2_ref, vh_ref, vct_ref):
    h1, c1 = _unpack_words(g1_ref[...])
    h2, c2 = _unpack_words(g2_ref[...])
    hs = h1 + h2
    cd = c1 - c2
    radial = jnp.sum(cd * cd, axis=1, keepdims=True)
    t = _silu(hs + radial * wr_ref[...] + b1_ref[...])
    ef = _silu(_mmt(t, w2_ref[...]) + b2_ref[...])
    cm = _silu(_mmt(ef, cw1_ref[...]) + cb1_ref[...])
    cms = _mmt(cm, cw2_ref[...])
    vh_ref[...] = ef
    t8 = jnp.concatenate([cd * cms, jnp.ones((BE, 1), jnp.float32),
                          jnp.zeros((BE, 4), jnp.float32)], axis=1)
    vct_ref[...] = jnp.swapaxes(t8.reshape(BE // CH, CH, 8), 1, 2)


def _edge_stage(g1, g2, wr, b1, w2, b2, cw1, cb1, cw2):
    wspec = lambda shape: pl.BlockSpec(shape, lambda i: (0, 0))
    return pl.pallas_call(
        _edge_body,
        grid=(E // BE,),
        in_specs=[
            pl.BlockSpec((BE, D), lambda i: (i, 0)),
            pl.BlockSpec((BE, D), lambda i: (i, 0)),
            wspec((1, D)), wspec((1, D)), wspec((D, D)), wspec((1, D)),
            wspec((D, D)), wspec((1, D)), wspec((1, D)),
        ],
        out_specs=[
            pl.BlockSpec((BE, D), lambda i: (i, 0)),
            pl.BlockSpec((BE // CH, 8, CH), lambda i: (i, 0, 0)),
        ],
        out_shape=[
            jax.ShapeDtypeStruct((E, D), jnp.float32),
            jax.ShapeDtypeStruct((E // CH, 8, CH), jnp.float32),
        ],
    )(g1, g2, wr, b1, w2, b2, cw1, cb1, cw2)


BN = 1024


def _pack_tables(h_new, coord_new, wa, wb):
    t1 = _pack_words(_mmt(h_new, wa), coord_new)
    t2 = _pack_words(_mmt(h_new, wb), coord_new)
    return t1, t2


def _node_common(p0, p1, c0, c1, cp, h, nw1a, nw1b, nb1, nw2, nb2):
    agg = p0 + p1
    sc = jnp.swapaxes(c0 + c1, 0, 1)
    cnt = jnp.maximum(sc[:, 3:4], 1.0)
    coord_new = cp[:, :3] + sc[:, :3] / cnt
    m = _silu(_mmt(h, nw1a) + _mmt(agg, nw1b) + nb1)
    h_new = h + _mmt(m, nw2) + nb2
    return h_new, coord_new


def _node_body(p0_ref, p1_ref, c0_ref, c1_ref, h_ref, cp_ref,
               nw1a_ref, nw1b_ref, nb1_ref, nw2_ref, nb2_ref,
               wa_ref, wb_ref, h_out, cp_out, t1_out, t2_out):
    h_new, coord_new = _node_common(
        p0_ref[...], p1_ref[...], c0_ref[...], c1_ref[...], cp_ref[...],
        h_ref[...], nw1a_ref[...], nw1b_ref[...], nb1_ref[...],
        nw2_ref[...], nb2_ref[...])
    h_out[...] = h_new
    cp_out[...] = jnp.concatenate(
        [coord_new, jnp.zeros((BN, CP - 3), jnp.float32)], axis=1)
    t1, t2 = _pack_tables(h_new, coord_new, wa_ref[...], wb_ref[...])
    t1_out[...] = t1
    t2_out[...] = t2


def _node_stage(p0, p1, c0, c1, h, cpad, nw1a, nw1b, nb1, nw2, nb2, wa, wb):
    wspec = lambda shape: pl.BlockSpec(shape, lambda i: (0, 0))
    bspec = lambda w: pl.BlockSpec((BN, w), lambda i: (i, 0))
    cspec = pl.BlockSpec((4, BN), lambda i: (0, i))
    return pl.pallas_call(
        _node_body,
        grid=(NP // BN,),
        in_specs=[
            bspec(D), bspec(D), cspec, cspec, bspec(D), bspec(CP),
            wspec((D, D)), wspec((D, D)), wspec((1, D)),
            wspec((D, D)), wspec((1, D)), wspec((D, D)), wspec((D, D)),
        ],
        out_specs=[
            bspec(D), bspec(CP), bspec(D), bspec(D),
        ],
        out_shape=[
            jax.ShapeDtypeStruct((NP, D), jnp.float32),
            jax.ShapeDtypeStruct((NP, CP), jnp.float32),
            jax.ShapeDtypeStruct((NP, D), jnp.int32),
            jax.ShapeDtypeStruct((NP, D), jnp.int32),
        ],
    )(p0, p1, c0, c1, h, cpad, nw1a, nw1b, nb1, nw2, nb2, wa, wb)


def _final_body(p0_ref, p1_ref, c0_ref, c1_ref, h_ref, cp_ref,
                nw1a_ref, nw1b_ref, nb1_ref, nw2_ref, nb2_ref,
                ow_ref, ob_ref, h_out, c_out):
    h_new, coord_new = _node_common(
        p0_ref[...], p1_ref[...], c0_ref[...], c1_ref[...], cp_ref[...],
        h_ref[...], nw1a_ref[...], nw1b_ref[...], nb1_ref[...],
        nw2_ref[...], nb2_ref[...])
    h_out[...] = _mmt(h_new, ow_ref[...]) + ob_ref[...]
    c_out[...] = jnp.concatenate(
        [coord_new, jnp.zeros((BN, CP - 3), jnp.float32)], axis=1)


def _final_stage(p0, p1, c0, c1, h, cpad, nw1a, nw1b, nb1, nw2, nb2, ow, ob):
    wspec = lambda shape: pl.BlockSpec(shape, lambda i: (0, 0))
    bspec = lambda w: pl.BlockSpec((BN, w), lambda i: (i, 0))
    cspec = pl.BlockSpec((4, BN), lambda i: (0, i))
    return pl.pallas_call(
        _final_body,
        grid=(NP // BN,),
        in_specs=[
            bspec(D), bspec(D), cspec, cspec, bspec(D), bspec(CP),
            wspec((D, D)), wspec((D, D)), wspec((1, D)),
            wspec((D, D)), wspec((1, D)), wspec((D, D)), wspec((1, D)),
        ],
        out_specs=[bspec(D), bspec(CP)],
        out_shape=[
            jax.ShapeDtypeStruct((NP, D), jnp.float32),
            jax.ShapeDtypeStruct((NP, CP), jnp.float32),
        ],
    )(p0, p1, c0, c1, h, cpad, nw1a, nw1b, nb1, nw2, nb2, ow, ob)


def _prep_body(h_ref, cp_ref, wa_ref, wb_ref, t1_out, t2_out):
    t1, t2 = _pack_tables(h_ref[...], cp_ref[:, :3], wa_ref[...], wb_ref[...])
    t1_out[...] = t1
    t2_out[...] = t2


def _prep_stage(h, cpad, wa, wb):
    wspec = lambda shape: pl.BlockSpec(shape, lambda i: (0, 0))
    return pl.pallas_call(
        _prep_body,
        grid=(NP // BN,),
        in_specs=[
            pl.BlockSpec((BN, D), lambda i: (i, 0)),
            pl.BlockSpec((BN, CP), lambda i: (i, 0)),
            wspec((D, D)), wspec((D, D)),
        ],
        out_specs=[
            pl.BlockSpec((BN, D), lambda i: (i, 0)),
            pl.BlockSpec((BN, D), lambda i: (i, 0)),
        ],
        out_shape=[
            jax.ShapeDtypeStruct((NP, D), jnp.int32),
            jax.ShapeDtypeStruct((NP, D), jnp.int32),
        ],
    )(h, cpad, wa, wb)


def kernel(h, x, edges, params):
    row = edges[0]
    col = edges[1]
    h_pad = jnp.pad(h, ((0, NP - N), (0, 0)))
    cpad = jnp.pad(x, ((0, NP - N), (0, CP - 3)))
    zeros = jnp.zeros((NP, D), jnp.float32)
    zeros_c = jnp.zeros((4 * NP,), jnp.float32)

    def layer_w(i):
        ew1 = params[f"ew1_{i}"]
        wa = ew1[:, :D]
        wb = ew1[:, D:2 * D]
        wr = ew1[:, 2 * D:].reshape(1, D)
        b1 = params[f"eb1_{i}"].reshape(1, D)
        w2 = params[f"ew2_{i}"]
        b2 = params[f"eb2_{i}"].reshape(1, D)
        nw1 = params[f"nw1_{i}"]
        nw1a = nw1[:, :D]
        nw1b = nw1[:, D:]
        nb1 = params[f"nb1_{i}"].reshape(1, D)
        nw2 = params[f"nw2_{i}"]
        nb2 = params[f"nb2_{i}"].reshape(1, D)
        cw1 = params[f"cw1_{i}"]
        cb1 = params[f"cb1_{i}"].reshape(1, D)
        cw2 = params[f"cw2_{i}"]
        return wa, wb, wr, b1, w2, b2, nw1a, nw1b, nb1, nw2, nb2, cw1, cb1, cw2

    wa0, wb0 = layer_w(0)[:2]
    t1, t2 = _prep_stage(h_pad, cpad, wa0, wb0)

    for i in range(L):
        wa, wb, wr, b1, w2, b2, nw1a, nw1b, nb1, nw2, nb2, cw1, cb1, cw2 = layer_w(i)
        g1, g2 = _sc_gather(t1, t2, row, col)
        val_h, vct = _edge_stage(g1, g2, wr, b1, w2, b2, cw1, cb1, cw2)
        p0, p1, c0f, c1f = _sc_scatter(val_h, vct, row, zeros, zeros_c)
        c0 = c0f.reshape(4, NP)
        c1 = c1f.reshape(4, NP)
        if i < L - 1:
            wa_n, wb_n = layer_w(i + 1)[:2]
            h_pad, cpad, t1, t2 = _node_stage(
                p0, p1, c0, c1, h_pad, cpad, nw1a, nw1b, nb1, nw2, nb2,
                wa_n, wb_n)
        else:
            h_fin, c_fin = _final_stage(
                p0, p1, c0, c1, h_pad, cpad, nw1a, nw1b, nb1, nw2, nb2,
                params["out_w"], params["out_b"].reshape(1, D))

    return (h_fin[:N], c_fin[:N, :3])


# speedup vs baseline: 5.0361x; 1.1843x over previous
import jax
import jax.numpy as jnp
from jax import lax
from jax.experimental import pallas as pl
from jax.experimental.pallas import tpu as pltpu
from jax.experimental.pallas import tpu_sc as plsc

N = 10000
E = 320000
D = 128
L = 4
CP = 16
NP = 10240
NC = 2
NS = 16
NW = NC * NS
CH = 128
NCHUNK = E // CH
FULL_W = NCHUNK // NW
EXTRA_W = NCHUNK - FULL_W * NW
NCHUNK_SC = NCHUNK // NC
FULL_T = NCHUNK_SC // NS
EXTRA_T = NCHUNK_SC - FULL_T * NS


def _sc_gather_body(t1, t2, row, col, g1, g2, idx_r, idx_c, buf1, buf2, s1, s2):
    wid = lax.axis_index("s") * NC + lax.axis_index("c")

    def do_chunk(ci):
        base = ci * CH
        pltpu.sync_copy(row.at[pl.ds(base, CH)], idx_r)
        pltpu.sync_copy(col.at[pl.ds(base, CH)], idx_c)
        cp1 = pltpu.async_copy(t1.at[idx_r], buf1, s1)
        cp2 = pltpu.async_copy(t2.at[idx_c], buf2, s2)
        cp1.wait()
        cp2.wait()
        pltpu.sync_copy(buf1, g1.at[pl.ds(base, CH)])
        pltpu.sync_copy(buf2, g2.at[pl.ds(base, CH)])

    def body(k, carry):
        do_chunk(wid + k * NW)
        return carry

    lax.fori_loop(0, FULL_W, body, 0)

    @pl.when(wid < EXTRA_W)
    def _():
        do_chunk(wid + FULL_W * NW)


NPAIR = FULL_T // 2


def _sc_scatter_body(val, vct, row, zeros, zeros_c, out0, out1, cout0, cout1,
                     idxA, idxB, e1A, e2A, e3A, e1B, e2B, e3B,
                     bufA, bufB, tbA, tbB, acc, accc, sLA, sLB, sAA, sAB):
    cid = lax.axis_index("c")
    sid = lax.axis_index("s")
    rs = NP // NS
    cs = (4 * NP) // NS

    pltpu.sync_copy(zeros.at[pl.ds(sid * rs, rs)], acc.at[pl.ds(sid * rs, rs)])
    pltpu.sync_copy(zeros_c.at[pl.ds(sid * cs, cs)],
                    accc.at[pl.ds(sid * cs, cs)])
    plsc.subcore_barrier()

    tbase = cid * NCHUNK_SC + sid
    setA = (idxA, e1A, e2A, e3A, bufA, tbA, sLA, sAA)
    setB = (idxB, e1B, e2B, e3B, bufB, tbB, sLB, sAB)

    def loads(m, s):
        idxv, e1, e2, e3, buf, tb, sL, sA = s
        c = tbase + m * NS
        base = c * CH
        pltpu.async_copy(row.at[pl.ds(base, CH)], idxv, sL)
        pltpu.async_copy(val.at[pl.ds(base, CH)], buf, sL)
        pltpu.async_copy(vct.at[c], tb, sL)

    def wait_loads(s):
        idxv, e1, e2, e3, buf, tb, sL, sA = s
        pltpu.make_async_copy(row.at[pl.ds(0, CH)], idxv, sL).wait()
        pltpu.make_async_copy(val.at[pl.ds(0, CH)], buf, sL).wait()
        pltpu.make_async_copy(vct.at[0], tb, sL).wait()

    def adds(s):
        idxv, e1, e2, e3, buf, tb, sL, sA = s
        for j in range(8):
            sl = pl.ds(j * 16, 16)
            v = idxv[sl]
            e1[sl] = v + NP
            e2[sl] = v + 2 * NP
            e3[sl] = v + 3 * NP
        pltpu.async_copy(buf, acc.at[idxv], sA, add=True)
        pltpu.async_copy(tb.at[0], accc.at[idxv], sA, add=True)
        pltpu.async_copy(tb.at[1], accc.at[e1], sA, add=True)
        pltpu.async_copy(tb.at[2], accc.at[e2], sA, add=True)
        pltpu.async_copy(tb.at[3], accc.at[e3], sA, add=True)

    def wait_adds(s):
        idxv, e1, e2, e3, buf, tb, sL, sA = s
        pltpu.make_async_copy(buf, acc.at[idxv], sA).wait()
        pltpu.make_async_copy(tb.at[0], accc.at[idxv], sA).wait()
        pltpu.make_async_copy(tb.at[1], accc.at[e1], sA).wait()
        pltpu.make_async_copy(tb.at[2], accc.at[e2], sA).wait()
        pltpu.make_async_copy(tb.at[3], accc.at[e3], sA).wait()

    loads(0, setA)

    def body(k, carry):
        wait_loads(setA)

        @pl.when(k > 0)
        def _():
            wait_adds(setB)

        loads(2 * k + 1, setB)
        adds(setA)
        wait_loads(setB)
        wait_adds(setA)

        @pl.when(k < NPAIR - 1)
        def _():
            loads(2 * k + 2, setA)

        adds(setB)
        return carry

    lax.fori_loop(0, NPAIR, body, 0)
    wait_adds(setB)

    @pl.when(sid < EXTRA_T)
    def _():
        c = tbase + FULL_T * NS
        base = c * CH
        pltpu.sync_copy(row.at[pl.ds(base, CH)], idxA)
        pltpu.sync_copy(val.at[pl.ds(base, CH)], bufA)
        pltpu.sync_copy(bufA, acc.at[idxA], add=True)
        pltpu.sync_copy(vct.at[c], tbA)
        for j in range(8):
            sl = pl.ds(j * 16, 16)
            v = idxA[sl]
            e1A[sl] = v + NP
            e2A[sl] = v + 2 * NP
            e3A[sl] = v + 3 * NP
        pltpu.sync_copy(tbA.at[0], accc.at[idxA], add=True)
        pltpu.sync_copy(tbA.at[1], accc.at[e1A], add=True)
        pltpu.sync_copy(tbA.at[2], accc.at[e2A], add=True)
        pltpu.sync_copy(tbA.at[3], accc.at[e3A], add=True)

    plsc.subcore_barrier()

    @pl.when(cid == 0)
    def _():
        pltpu.sync_copy(acc.at[pl.ds(sid * rs, rs)], out0.at[pl.ds(sid * rs, rs)])
        pltpu.sync_copy(accc.at[pl.ds(sid * cs, cs)],
                        cout0.at[pl.ds(sid * cs, cs)])

    @pl.when(cid == 1)
    def _():
        pltpu.sync_copy(acc.at[pl.ds(sid * rs, rs)], out1.at[pl.ds(sid * rs, rs)])
        pltpu.sync_copy(accc.at[pl.ds(sid * cs, cs)],
                        cout1.at[pl.ds(sid * cs, cs)])


_sc_cache = {}


def _sc_gather(t1, t2, row, col):
    if "gather" not in _sc_cache:
        mesh = plsc.VectorSubcoreMesh(core_axis_name="c", subcore_axis_name="s")
        _sc_cache["gather"] = pl.kernel(
            _sc_gather_body,
            mesh=mesh,
            out_type=(
                jax.ShapeDtypeStruct((E, D), jnp.int32),
                jax.ShapeDtypeStruct((E, D), jnp.int32),
            ),
            scratch_types=[
                pltpu.VMEM((CH,), jnp.int32),
                pltpu.VMEM((CH,), jnp.int32),
                pltpu.VMEM((CH, D), jnp.int32),
                pltpu.VMEM((CH, D), jnp.int32),
                pltpu.SemaphoreType.DMA,
                pltpu.SemaphoreType.DMA,
            ],
        )
    return _sc_cache["gather"](t1, t2, row, col)


def _sc_scatter(val, vct, row, zeros, zeros_c):
    if "scatter" not in _sc_cache:
        mesh = plsc.VectorSubcoreMesh(core_axis_name="c", subcore_axis_name="s")
        _sc_cache["scatter"] = pl.kernel(
            _sc_scatter_body,
            mesh=mesh,
            out_type=(
                jax.ShapeDtypeStruct((NP, D), jnp.float32),
                jax.ShapeDtypeStruct((NP, D), jnp.float32),
                jax.ShapeDtypeStruct((4 * NP,), jnp.float32),
                jax.ShapeDtypeStruct((4 * NP,), jnp.float32),
            ),
            scratch_types=(
                [pltpu.VMEM((CH,), jnp.int32) for _ in range(8)]
                + [pltpu.VMEM((CH, D), jnp.float32) for _ in range(2)]
                + [pltpu.VMEM((8, CH), jnp.float32) for _ in range(2)]
                + [pltpu.VMEM_SHARED((NP, D), jnp.float32),
                   pltpu.VMEM_SHARED((4 * NP,), jnp.float32)]
                + [pltpu.SemaphoreType.DMA for _ in range(4)]
            ),
        )
    return _sc_cache["scatter"](val, vct, row, zeros, zeros_c)


def _pack_words(payload, coord3):
    pay = lax.convert_element_type(
        lax.bitcast_convert_type(payload.astype(jnp.bfloat16), jnp.uint16),
        jnp.uint32)
    cbits = lax.bitcast_convert_type(coord3, jnp.uint32)
    hi = cbits >> 16
    lo = cbits & 0xFFFF
    b = payload.shape[0]
    z13 = jnp.zeros((b, 13), jnp.uint32)
    z109 = jnp.zeros((b, 109), jnp.uint32)
    cplane = jnp.concatenate([hi, z13, lo, z109], axis=1)
    return lax.bitcast_convert_type(pay | (cplane << 16), jnp.int32)


def _unpack_words(words):
    w = lax.bitcast_convert_type(words, jnp.uint32)
    pay = lax.bitcast_convert_type(
        lax.convert_element_type(w & 0xFFFF, jnp.uint16),
        jnp.bfloat16).astype(jnp.float32)
    cplane = w >> 16
    coord = lax.bitcast_convert_type(
        (cplane[:, 0:3] << 16) | cplane[:, 16:19], jnp.float32)
    return pay, coord


BE = 3200


def _sigmoid(x):
    return 1.0 / (1.0 + jnp.exp(-x))


def _silu(x):
    return x * _sigmoid(x)


def _mmt(x, w):
    return lax.dot_general(x, w, (((1,), (1,)), ((), ())),
                           preferred_element_type=jnp.float32)


def _edge_body(g1_ref, g2_ref, wr_ref, b1_ref, w2_ref, b2_ref,
               cw1_ref, cb1_ref, cw2_ref, vh_ref, vct_ref):
    h1, c1 = _unpack_words(g1_ref[...])
    h2, c2 = _unpack_words(g2_ref[...])
    hs = h1 + h2
    cd = c1 - c2
    radial = jnp.sum(cd * cd, axis=1, keepdims=True)
    t = _silu(hs + radial * wr_ref[...] + b1_ref[...])
    ef = _silu(_mmt(t, w2_ref[...]) + b2_ref[...])
    cm = _silu(_mmt(ef, cw1_ref[...]) + cb1_ref[...])
    cms = _mmt(cm, cw2_ref[...])
    vh_ref[...] = ef
    t8 = jnp.concatenate([cd * cms, jnp.ones((BE, 1), jnp.float32),
                          jnp.zeros((BE, 4), jnp.float32)], axis=1)
    vct_ref[...] = jnp.swapaxes(t8.reshape(BE // CH, CH, 8), 1, 2)


def _edge_stage(g1, g2, wr, b1, w2, b2, cw1, cb1, cw2):
    wspec = lambda shape: pl.BlockSpec(shape, lambda i: (0, 0))
    return pl.pallas_call(
        _edge_body,
        grid=(E // BE,),
        in_specs=[
            pl.BlockSpec((BE, D), lambda i: (i, 0)),
            pl.BlockSpec((BE, D), lambda i: (i, 0)),
            wspec((1, D)), wspec((1, D)), wspec((D, D)), wspec((1, D)),
            wspec((D, D)), wspec((1, D)), wspec((1, D)),
        ],
        out_specs=[
            pl.BlockSpec((BE, D), lambda i: (i, 0)),
            pl.BlockSpec((BE // CH, 8, CH), lambda i: (i, 0, 0)),
        ],
        out_shape=[
            jax.ShapeDtypeStruct((E, D), jnp.float32),
            jax.ShapeDtypeStruct((E // CH, 8, CH), jnp.float32),
        ],
    )(g1, g2, wr, b1, w2, b2, cw1, cb1, cw2)


BN = 1024


def _node_common(p0, p1, c0, c1, ct, h, nw1a, nw1b, nb1, nw2, nb2):
    agg = p0 + p1
    sc = jnp.swapaxes(c0 + c1, 0, 1)
    cnt = jnp.maximum(sc[:, 3:4], 1.0)
    cold = jnp.swapaxes(ct, 0, 1)[:, :3]
    coord_new = cold + sc[:, :3] / cnt
    m = _silu(_mmt(h, nw1a) + _mmt(agg, nw1b) + nb1)
    h_new = h + _mmt(m, nw2) + nb2
    return h_new, coord_new


def _ctab_block(coord_new):
    b = coord_new.shape[0]
    return jnp.swapaxes(jnp.concatenate(
        [coord_new, jnp.zeros((b, 1), jnp.float32)], axis=1), 0, 1)


def _node_body(p0_ref, p1_ref, c0_ref, c1_ref, h_ref, ct_ref,
               nw1a_ref, nw1b_ref, nb1_ref, nw2_ref, nb2_ref,
               wa_ref, wb_ref, h_out, ct_out, t1_out, t2_out):
    h_new, coord_new = _node_common(
        p0_ref[...], p1_ref[...], c0_ref[...], c1_ref[...], ct_ref[...],
        h_ref[...], nw1a_ref[...], nw1b_ref[...], nb1_ref[...],
        nw2_ref[...], nb2_ref[...])
    h_out[...] = h_new
    ct_out[...] = _ctab_block(coord_new)
    t1_out[...] = _pack_words(_mmt(h_new, wa_ref[...]), coord_new)
    t2_out[...] = _pack_words(_mmt(h_new, wb_ref[...]), coord_new)


def _node_stage(p0, p1, c0, c1, h, ctab, nw1a, nw1b, nb1, nw2, nb2, wa, wb):
    wspec = lambda shape: pl.BlockSpec(shape, lambda i: (0, 0))
    bspec = lambda w: pl.BlockSpec((BN, w), lambda i: (i, 0))
    cspec = pl.BlockSpec((4, BN), lambda i: (0, i))
    return pl.pallas_call(
        _node_body,
        grid=(NP // BN,),
        in_specs=[
            bspec(D), bspec(D), cspec, cspec, bspec(D), cspec,
            wspec((D, D)), wspec((D, D)), wspec((1, D)),
            wspec((D, D)), wspec((1, D)), wspec((D, D)), wspec((D, D)),
        ],
        out_specs=[
            bspec(D), cspec, bspec(D), bspec(D),
        ],
        out_shape=[
            jax.ShapeDtypeStruct((NP, D), jnp.float32),
            jax.ShapeDtypeStruct((4, NP), jnp.float32),
            jax.ShapeDtypeStruct((NP, D), jnp.int32),
            jax.ShapeDtypeStruct((NP, D), jnp.int32),
        ],
    )(p0, p1, c0, c1, h, ctab, nw1a, nw1b, nb1, nw2, nb2, wa, wb)


def _final_body(p0_ref, p1_ref, c0_ref, c1_ref, h_ref, ct_ref,
                nw1a_ref, nw1b_ref, nb1_ref, nw2_ref, nb2_ref,
                ow_ref, ob_ref, h_out, c_out):
    h_new, coord_new = _node_common(
        p0_ref[...], p1_ref[...], c0_ref[...], c1_ref[...], ct_ref[...],
        h_ref[...], nw1a_ref[...], nw1b_ref[...], nb1_ref[...],
        nw2_ref[...], nb2_ref[...])
    h_out[...] = _mmt(h_new, ow_ref[...]) + ob_ref[...]
    c_out[...] = jnp.concatenate(
        [coord_new, jnp.zeros((BN, CP - 3), jnp.float32)], axis=1)


def _final_stage(p0, p1, c0, c1, h, ctab, nw1a, nw1b, nb1, nw2, nb2, ow, ob):
    wspec = lambda shape: pl.BlockSpec(shape, lambda i: (0, 0))
    bspec = lambda w: pl.BlockSpec((BN, w), lambda i: (i, 0))
    cspec = pl.BlockSpec((4, BN), lambda i: (0, i))
    return pl.pallas_call(
        _final_body,
        grid=(NP // BN,),
        in_specs=[
            bspec(D), bspec(D), cspec, cspec, bspec(D), cspec,
            wspec((D, D)), wspec((D, D)), wspec((1, D)),
            wspec((D, D)), wspec((1, D)), wspec((D, D)), wspec((1, D)),
        ],
        out_specs=[bspec(D), bspec(CP)],
        out_shape=[
            jax.ShapeDtypeStruct((NP, D), jnp.float32),
            jax.ShapeDtypeStruct((NP, CP), jnp.float32),
        ],
    )(p0, p1, c0, c1, h, ctab, nw1a, nw1b, nb1, nw2, nb2, ow, ob)


def _prep_body(h_ref, ct_ref, wa_ref, wb_ref, t1_out, t2_out):
    coord3 = jnp.swapaxes(ct_ref[...], 0, 1)[:, :3]
    t1_out[...] = _pack_words(_mmt(h_ref[...], wa_ref[...]), coord3)
    t2_out[...] = _pack_words(_mmt(h_ref[...], wb_ref[...]), coord3)


def _prep_stage(h, ctab, wa, wb):
    wspec = lambda shape: pl.BlockSpec(shape, lambda i: (0, 0))
    return pl.pallas_call(
        _prep_body,
        grid=(NP // BN,),
        in_specs=[
            pl.BlockSpec((BN, D), lambda i: (i, 0)),
            pl.BlockSpec((4, BN), lambda i: (0, i)),
            wspec((D, D)), wspec((D, D)),
        ],
        out_specs=[
            pl.BlockSpec((BN, D), lambda i: (i, 0)),
            pl.BlockSpec((BN, D), lambda i: (i, 0)),
        ],
        out_shape=[
            jax.ShapeDtypeStruct((NP, D), jnp.int32),
            jax.ShapeDtypeStruct((NP, D), jnp.int32),
        ],
    )(h, ctab, wa, wb)


def kernel(h, x, edges, params):
    row = edges[0]
    col = edges[1]
    h_pad = jnp.pad(h, ((0, NP - N), (0, 0)))
    ctab = jnp.pad(x, ((0, NP - N), (0, 1))).T
    zeros = jnp.zeros((NP, D), jnp.float32)
    zeros_c = jnp.zeros((4 * NP,), jnp.float32)

    def layer_w(i):
        ew1 = params[f"ew1_{i}"]
        wa = ew1[:, :D]
        wb = ew1[:, D:2 * D]
        wr = ew1[:, 2 * D:].reshape(1, D)
        b1 = params[f"eb1_{i}"].reshape(1, D)
        w2 = params[f"ew2_{i}"]
        b2 = params[f"eb2_{i}"].reshape(1, D)
        nw1 = params[f"nw1_{i}"]
        nw1a = nw1[:, :D]
        nw1b = nw1[:, D:]
        nb1 = params[f"nb1_{i}"].reshape(1, D)
        nw2 = params[f"nw2_{i}"]
        nb2 = params[f"nb2_{i}"].reshape(1, D)
        cw1 = params[f"cw1_{i}"]
        cb1 = params[f"cb1_{i}"].reshape(1, D)
        cw2 = params[f"cw2_{i}"]
        return wa, wb, wr, b1, w2, b2, nw1a, nw1b, nb1, nw2, nb2, cw1, cb1, cw2

    wa0, wb0 = layer_w(0)[:2]
    t1, t2 = _prep_stage(h_pad, ctab, wa0, wb0)

    for i in range(L):
        wa, wb, wr, b1, w2, b2, nw1a, nw1b, nb1, nw2, nb2, cw1, cb1, cw2 = layer_w(i)
        g1, g2 = _sc_gather(t1, t2, row, col)
        val_h, vct = _edge_stage(g1, g2, wr, b1, w2, b2, cw1, cb1, cw2)
        p0, p1, c0f, c1f = _sc_scatter(val_h, vct, row, zeros, zeros_c)
        c0 = c0f.reshape(4, NP)
        c1 = c1f.reshape(4, NP)
        if i < L - 1:
            wa_n, wb_n = layer_w(i + 1)[:2]
            h_pad, ctab, t1, t2 = _node_stage(
                p0, p1, c0, c1, h_pad, ctab, nw1a, nw1b, nb1, nw2, nb2,
                wa_n, wb_n)
        else:
            h_fin, c_fin = _final_stage(
                p0, p1, c0, c1, h_pad, ctab, nw1a, nw1b, nb1, nw2, nb2,
                params["out_w"], params["out_b"].reshape(1, D))

    return (h_fin[:N], c_fin[:N, :3])


# speedup vs baseline: 5.8187x; 1.1554x over previous
import jax
import jax.numpy as jnp
from jax import lax
from jax.experimental import pallas as pl
from jax.experimental.pallas import tpu as pltpu
from jax.experimental.pallas import tpu_sc as plsc

N = 10000
E = 320000
D = 128
L = 4
CP = 16
NP = 10240
NC = 2
NS = 16
NW = NC * NS
CH = 128
NCHUNK = E // CH
FULL_W = NCHUNK // NW
EXTRA_W = NCHUNK - FULL_W * NW
NCHUNK_SC = NCHUNK // NC
FULL_T = NCHUNK_SC // NS
EXTRA_T = NCHUNK_SC - FULL_T * NS


NPAIR_W = FULL_W // 2


def _sc_gather_body(t1, t2, row, col, g1, g2,
                    irA, icA, irB, icB, b1A, b2A, b1B, b2B,
                    sIA, sIB, sGA, sGB, sWA, sWB):
    wid = lax.axis_index("s") * NC + lax.axis_index("c")

    def loads_i(m, s):
        irv, icv, b1, b2, sI, sG, sW = s
        base = (wid + m * NW) * CH
        pltpu.async_copy(row.at[pl.ds(base, CH)], irv, sI)
        pltpu.async_copy(col.at[pl.ds(base, CH)], icv, sI)

    def wait_i(s):
        irv, icv, b1, b2, sI, sG, sW = s
        pltpu.make_async_copy(row.at[pl.ds(0, CH)], irv, sI).wait()
        pltpu.make_async_copy(col.at[pl.ds(0, CH)], icv, sI).wait()

    def gathers(s):
        irv, icv, b1, b2, sI, sG, sW = s
        pltpu.async_copy(t1.at[irv], b1, sG)
        pltpu.async_copy(t2.at[icv], b2, sG)

    def wait_g(s):
        irv, icv, b1, b2, sI, sG, sW = s
        pltpu.make_async_copy(t1.at[irv], b1, sG).wait()
        pltpu.make_async_copy(t2.at[icv], b2, sG).wait()

    def wbs(m, s):
        irv, icv, b1, b2, sI, sG, sW = s
        base = (wid + m * NW) * CH
        pltpu.async_copy(b1, g1.at[pl.ds(base, CH)], sW)
        pltpu.async_copy(b2, g2.at[pl.ds(base, CH)], sW)

    def wait_w(s):
        irv, icv, b1, b2, sI, sG, sW = s
        pltpu.make_async_copy(b1, g1.at[pl.ds(0, CH)], sW).wait()
        pltpu.make_async_copy(b2, g2.at[pl.ds(0, CH)], sW).wait()

    setA = (irA, icA, b1A, b2A, sIA, sGA, sWA)
    setB = (irB, icB, b1B, b2B, sIB, sGB, sWB)

    loads_i(0, setA)

    def body(k, carry):
        wait_i(setA)

        @pl.when(k > 0)
        def _():
            wait_w(setA)

        gathers(setA)
        loads_i(2 * k + 1, setB)
        wait_g(setA)
        wbs(2 * k, setA)
        wait_i(setB)

        @pl.when(k > 0)
        def _():
            wait_w(setB)

        gathers(setB)

        @pl.when(k < NPAIR_W - 1)
        def _():
            loads_i(2 * k + 2, setA)

        wait_g(setB)
        wbs(2 * k + 1, setB)
        return carry

    lax.fori_loop(0, NPAIR_W, body, 0)
    wait_w(setA)
    wait_w(setB)

    @pl.when(wid < EXTRA_W)
    def _():
        base = (wid + FULL_W * NW) * CH
        pltpu.sync_copy(row.at[pl.ds(base, CH)], irA)
        pltpu.sync_copy(col.at[pl.ds(base, CH)], icA)
        cp1 = pltpu.async_copy(t1.at[irA], b1A, sGA)
        cp2 = pltpu.async_copy(t2.at[icA], b2A, sGB)
        cp1.wait()
        cp2.wait()
        pltpu.sync_copy(b1A, g1.at[pl.ds(base, CH)])
        pltpu.sync_copy(b2A, g2.at[pl.ds(base, CH)])


NPAIR = FULL_T // 2


def _sc_scatter_body(val, vct, row, zeros, zeros_c, out0, out1, cout0, cout1,
                     idxA, idxB, e1A, e2A, e3A, e1B, e2B, e3B,
                     bufA, bufB, tbA, tbB, acc, accc, sLA, sLB, sAA, sAB):
    cid = lax.axis_index("c")
    sid = lax.axis_index("s")
    rs = NP // NS
    cs = (4 * NP) // NS

    pltpu.sync_copy(zeros.at[pl.ds(sid * rs, rs)], acc.at[pl.ds(sid * rs, rs)])
    pltpu.sync_copy(zeros_c.at[pl.ds(sid * cs, cs)],
                    accc.at[pl.ds(sid * cs, cs)])
    plsc.subcore_barrier()

    tbase = cid * NCHUNK_SC + sid
    setA = (idxA, e1A, e2A, e3A, bufA, tbA, sLA, sAA)
    setB = (idxB, e1B, e2B, e3B, bufB, tbB, sLB, sAB)

    def loads(m, s):
        idxv, e1, e2, e3, buf, tb, sL, sA = s
        c = tbase + m * NS
        base = c * CH
        pltpu.async_copy(row.at[pl.ds(base, CH)], idxv, sL)
        pltpu.async_copy(val.at[pl.ds(base, CH)], buf, sL)
        pltpu.async_copy(vct.at[c], tb, sL)

    def wait_loads(s):
        idxv, e1, e2, e3, buf, tb, sL, sA = s
        pltpu.make_async_copy(row.at[pl.ds(0, CH)], idxv, sL).wait()
        pltpu.make_async_copy(val.at[pl.ds(0, CH)], buf, sL).wait()
        pltpu.make_async_copy(vct.at[0], tb, sL).wait()

    def adds(s):
        idxv, e1, e2, e3, buf, tb, sL, sA = s
        for j in range(8):
            sl = pl.ds(j * 16, 16)
            v = idxv[sl]
            e1[sl] = v + NP
            e2[sl] = v + 2 * NP
            e3[sl] = v + 3 * NP
        pltpu.async_copy(buf, acc.at[idxv], sA, add=True)
        pltpu.async_copy(tb.at[0], accc.at[idxv], sA, add=True)
        pltpu.async_copy(tb.at[1], accc.at[e1], sA, add=True)
        pltpu.async_copy(tb.at[2], accc.at[e2], sA, add=True)
        pltpu.async_copy(tb.at[3], accc.at[e3], sA, add=True)

    def wait_adds(s):
        idxv, e1, e2, e3, buf, tb, sL, sA = s
        pltpu.make_async_copy(buf, acc.at[idxv], sA).wait()
        pltpu.make_async_copy(tb.at[0], accc.at[idxv], sA).wait()
        pltpu.make_async_copy(tb.at[1], accc.at[e1], sA).wait()
        pltpu.make_async_copy(tb.at[2], accc.at[e2], sA).wait()
        pltpu.make_async_copy(tb.at[3], accc.at[e3], sA).wait()

    loads(0, setA)

    def body(k, carry):
        wait_loads(setA)

        @pl.when(k > 0)
        def _():
            wait_adds(setB)

        loads(2 * k + 1, setB)
        adds(setA)
        wait_loads(setB)
        wait_adds(setA)

        @pl.when(k < NPAIR - 1)
        def _():
            loads(2 * k + 2, setA)

        adds(setB)
        return carry

    lax.fori_loop(0, NPAIR, body, 0)
    wait_adds(setB)

    @pl.when(sid < EXTRA_T)
    def _():
        c = tbase + FULL_T * NS
        base = c * CH
        pltpu.sync_copy(row.at[pl.ds(base, CH)], idxA)
        pltpu.sync_copy(val.at[pl.ds(base, CH)], bufA)
        pltpu.sync_copy(bufA, acc.at[idxA], add=True)
        pltpu.sync_copy(vct.at[c], tbA)
        for j in range(8):
            sl = pl.ds(j * 16, 16)
            v = idxA[sl]
            e1A[sl] = v + NP
            e2A[sl] = v + 2 * NP
            e3A[sl] = v + 3 * NP
        pltpu.sync_copy(tbA.at[0], accc.at[idxA], add=True)
        pltpu.sync_copy(tbA.at[1], accc.at[e1A], add=True)
        pltpu.sync_copy(tbA.at[2], accc.at[e2A], add=True)
        pltpu.sync_copy(tbA.at[3], accc.at[e3A], add=True)

    plsc.subcore_barrier()

    @pl.when(cid == 0)
    def _():
        pltpu.sync_copy(acc.at[pl.ds(sid * rs, rs)], out0.at[pl.ds(sid * rs, rs)])
        pltpu.sync_copy(accc.at[pl.ds(sid * cs, cs)],
                        cout0.at[pl.ds(sid * cs, cs)])

    @pl.when(cid == 1)
    def _():
        pltpu.sync_copy(acc.at[pl.ds(sid * rs, rs)], out1.at[pl.ds(sid * rs, rs)])
        pltpu.sync_copy(accc.at[pl.ds(sid * cs, cs)],
                        cout1.at[pl.ds(sid * cs, cs)])


_sc_cache = {}


def _sc_gather(t1, t2, row, col):
    if "gather" not in _sc_cache:
        mesh = plsc.VectorSubcoreMesh(core_axis_name="c", subcore_axis_name="s")
        _sc_cache["gather"] = pl.kernel(
            _sc_gather_body,
            mesh=mesh,
            out_type=(
                jax.ShapeDtypeStruct((E, D), jnp.int32),
                jax.ShapeDtypeStruct((E, D), jnp.int32),
            ),
            scratch_types=(
                [pltpu.VMEM((CH,), jnp.int32) for _ in range(4)]
                + [pltpu.VMEM((CH, D), jnp.int32) for _ in range(4)]
                + [pltpu.SemaphoreType.DMA for _ in range(6)]
            ),
        )
    return _sc_cache["gather"](t1, t2, row, col)


def _sc_scatter(val, vct, row, zeros, zeros_c):
    if "scatter" not in _sc_cache:
        mesh = plsc.VectorSubcoreMesh(core_axis_name="c", subcore_axis_name="s")
        _sc_cache["scatter"] = pl.kernel(
            _sc_scatter_body,
            mesh=mesh,
            out_type=(
                jax.ShapeDtypeStruct((NP, D), jnp.float32),
                jax.ShapeDtypeStruct((NP, D), jnp.float32),
                jax.ShapeDtypeStruct((4 * NP,), jnp.float32),
                jax.ShapeDtypeStruct((4 * NP,), jnp.float32),
            ),
            scratch_types=(
                [pltpu.VMEM((CH,), jnp.int32) for _ in range(8)]
                + [pltpu.VMEM((CH, D), jnp.float32) for _ in range(2)]
                + [pltpu.VMEM((8, CH), jnp.float32) for _ in range(2)]
                + [pltpu.VMEM_SHARED((NP, D), jnp.float32),
                   pltpu.VMEM_SHARED((4 * NP,), jnp.float32)]
                + [pltpu.SemaphoreType.DMA for _ in range(4)]
            ),
        )
    return _sc_cache["scatter"](val, vct, row, zeros, zeros_c)


def _pack_words(payload, coord3):
    pay = lax.convert_element_type(
        lax.bitcast_convert_type(payload.astype(jnp.bfloat16), jnp.uint16),
        jnp.uint32)
    cbits = lax.bitcast_convert_type(coord3, jnp.uint32)
    hi = cbits >> 16
    lo = cbits & 0xFFFF
    b = payload.shape[0]
    z13 = jnp.zeros((b, 13), jnp.uint32)
    z109 = jnp.zeros((b, 109), jnp.uint32)
    cplane = jnp.concatenate([hi, z13, lo, z109], axis=1)
    return lax.bitcast_convert_type(pay | (cplane << 16), jnp.int32)


def _unpack_words(words):
    w = lax.bitcast_convert_type(words, jnp.uint32)
    pay = lax.bitcast_convert_type(
        lax.convert_element_type(w & 0xFFFF, jnp.uint16),
        jnp.bfloat16).astype(jnp.float32)
    cplane = w >> 16
    coord = lax.bitcast_convert_type(
        (cplane[:, 0:3] << 16) | cplane[:, 16:19], jnp.float32)
    return pay, coord


BE = 3200


def _sigmoid(x):
    return 1.0 / (1.0 + jnp.exp(-x))


def _silu(x):
    return x * _sigmoid(x)


def _mmt(x, w):
    return lax.dot_general(x, w, (((1,), (1,)), ((), ())),
                           preferred_element_type=jnp.float32)


def _edge_body(g1_ref, g2_ref, wr_ref, b1_ref, w2_ref, b2_ref,
               cw1_ref, cb1_ref, cw2_ref, vh_ref, vct_ref):
    h1, c1 = _unpack_words(g1_ref[...])
    h2, c2 = _unpack_words(g2_ref[...])
    hs = h1 + h2
    cd = c1 - c2
    radial = jnp.sum(cd * cd, axis=1, keepdims=True)
    t = _silu(hs + radial * wr_ref[...] + b1_ref[...])
    ef = _silu(_mmt(t, w2_ref[...]) + b2_ref[...])
    cm = _silu(_mmt(ef, cw1_ref[...]) + cb1_ref[...])
    cms = _mmt(cm, cw2_ref[...])
    vh_ref[...] = ef
    t8 = jnp.concatenate([cd * cms, jnp.ones((BE, 1), jnp.float32),
                          jnp.zeros((BE, 4), jnp.float32)], axis=1)
    vct_ref[...] = jnp.swapaxes(t8.reshape(BE // CH, CH, 8), 1, 2)


def _edge_stage(g1, g2, wr, b1, w2, b2, cw1, cb1, cw2):
    wspec = lambda shape: pl.BlockSpec(shape, lambda i: (0, 0))
    return pl.pallas_call(
        _edge_body,
        grid=(E // BE,),
        in_specs=[
            pl.BlockSpec((BE, D), lambda i: (i, 0)),
            pl.BlockSpec((BE, D), lambda i: (i, 0)),
            wspec((1, D)), wspec((1, D)), wspec((D, D)), wspec((1, D)),
            wspec((D, D)), wspec((1, D)), wspec((1, D)),
        ],
        out_specs=[
            pl.BlockSpec((BE, D), lambda i: (i, 0)),
            pl.BlockSpec((BE // CH, 8, CH), lambda i: (i, 0, 0)),
        ],
        out_shape=[
            jax.ShapeDtypeStruct((E, D), jnp.float32),
            jax.ShapeDtypeStruct((E // CH, 8, CH), jnp.float32),
        ],
    )(g1, g2, wr, b1, w2, b2, cw1, cb1, cw2)


BN = 1024


def _node_common(p0, p1, c0, c1, ct, h, nw1a, nw1b, nb1, nw2, nb2):
    agg = p0 + p1
    sc = jnp.swapaxes(c0 + c1, 0, 1)
    cnt = jnp.maximum(sc[:, 3:4], 1.0)
    cold = jnp.swapaxes(ct, 0, 1)[:, :3]
    coord_new = cold + sc[:, :3] / cnt
    m = _silu(_mmt(h, nw1a) + _mmt(agg, nw1b) + nb1)
    h_new = h + _mmt(m, nw2) + nb2
    return h_new, coord_new


def _ctab_block(coord_new):
    b = coord_new.shape[0]
    return jnp.swapaxes(jnp.concatenate(
        [coord_new, jnp.zeros((b, 1), jnp.float32)], axis=1), 0, 1)


def _node_body(p0_ref, p1_ref, c0_ref, c1_ref, h_ref, ct_ref,
               nw1a_ref, nw1b_ref, nb1_ref, nw2_ref, nb2_ref,
               wa_ref, wb_ref, h_out, ct_out, t1_out, t2_out):
    h_new, coord_new = _node_common(
        p0_ref[...], p1_ref[...], c0_ref[...], c1_ref[...], ct_ref[...],
        h_ref[...], nw1a_ref[...], nw1b_ref[...], nb1_ref[...],
        nw2_ref[...], nb2_ref[...])
    h_out[...] = h_new
    ct_out[...] = _ctab_block(coord_new)
    t1_out[...] = _pack_words(_mmt(h_new, wa_ref[...]), coord_new)
    t2_out[...] = _pack_words(_mmt(h_new, wb_ref[...]), coord_new)


def _node_stage(p0, p1, c0, c1, h, ctab, nw1a, nw1b, nb1, nw2, nb2, wa, wb):
    wspec = lambda shape: pl.BlockSpec(shape, lambda i: (0, 0))
    bspec = lambda w: pl.BlockSpec((BN, w), lambda i: (i, 0))
    cspec = pl.BlockSpec((4, BN), lambda i: (0, i))
    return pl.pallas_call(
        _node_body,
        grid=(NP // BN,),
        in_specs=[
            bspec(D), bspec(D), cspec, cspec, bspec(D), cspec,
            wspec((D, D)), wspec((D, D)), wspec((1, D)),
            wspec((D, D)), wspec((1, D)), wspec((D, D)), wspec((D, D)),
        ],
        out_specs=[
            bspec(D), cspec, bspec(D), bspec(D),
        ],
        out_shape=[
            jax.ShapeDtypeStruct((NP, D), jnp.float32),
            jax.ShapeDtypeStruct((4, NP), jnp.float32),
            jax.ShapeDtypeStruct((NP, D), jnp.int32),
            jax.ShapeDtypeStruct((NP, D), jnp.int32),
        ],
    )(p0, p1, c0, c1, h, ctab, nw1a, nw1b, nb1, nw2, nb2, wa, wb)


def _final_body(p0_ref, p1_ref, c0_ref, c1_ref, h_ref, ct_ref,
                nw1a_ref, nw1b_ref, nb1_ref, nw2_ref, nb2_ref,
                ow_ref, ob_ref, h_out, c_out):
    h_new, coord_new = _node_common(
        p0_ref[...], p1_ref[...], c0_ref[...], c1_ref[...], ct_ref[...],
        h_ref[...], nw1a_ref[...], nw1b_ref[...], nb1_ref[...],
        nw2_ref[...], nb2_ref[...])
    h_out[...] = _mmt(h_new, ow_ref[...]) + ob_ref[...]
    c_out[...] = jnp.concatenate(
        [coord_new, jnp.zeros((BN, CP - 3), jnp.float32)], axis=1)


def _final_stage(p0, p1, c0, c1, h, ctab, nw1a, nw1b, nb1, nw2, nb2, ow, ob):
    wspec = lambda shape: pl.BlockSpec(shape, lambda i: (0, 0))
    bspec = lambda w: pl.BlockSpec((BN, w), lambda i: (i, 0))
    cspec = pl.BlockSpec((4, BN), lambda i: (0, i))
    return pl.pallas_call(
        _final_body,
        grid=(NP // BN,),
        in_specs=[
            bspec(D), bspec(D), cspec, cspec, bspec(D), cspec,
            wspec((D, D)), wspec((D, D)), wspec((1, D)),
            wspec((D, D)), wspec((1, D)), wspec((D, D)), wspec((1, D)),
        ],
        out_specs=[bspec(D), bspec(CP)],
        out_shape=[
            jax.ShapeDtypeStruct((NP, D), jnp.float32),
            jax.ShapeDtypeStruct((NP, CP), jnp.float32),
        ],
    )(p0, p1, c0, c1, h, ctab, nw1a, nw1b, nb1, nw2, nb2, ow, ob)


def _prep_body(h_ref, ct_ref, wa_ref, wb_ref, t1_out, t2_out):
    coord3 = jnp.swapaxes(ct_ref[...], 0, 1)[:, :3]
    t1_out[...] = _pack_words(_mmt(h_ref[...], wa_ref[...]), coord3)
    t2_out[...] = _pack_words(_mmt(h_ref[...], wb_ref[...]), coord3)


def _prep_stage(h, ctab, wa, wb):
    wspec = lambda shape: pl.BlockSpec(shape, lambda i: (0, 0))
    return pl.pallas_call(
        _prep_body,
        grid=(NP // BN,),
        in_specs=[
            pl.BlockSpec((BN, D), lambda i: (i, 0)),
            pl.BlockSpec((4, BN), lambda i: (0, i)),
            wspec((D, D)), wspec((D, D)),
        ],
        out_specs=[
            pl.BlockSpec((BN, D), lambda i: (i, 0)),
            pl.BlockSpec((BN, D), lambda i: (i, 0)),
        ],
        out_shape=[
            jax.ShapeDtypeStruct((NP, D), jnp.int32),
            jax.ShapeDtypeStruct((NP, D), jnp.int32),
        ],
    )(h, ctab, wa, wb)


def kernel(h, x, edges, params):
    row = edges[0]
    col = edges[1]
    h_pad = jnp.pad(h, ((0, NP - N), (0, 0)))
    ctab = jnp.pad(x, ((0, NP - N), (0, 1))).T
    zeros = jnp.zeros((NP, D), jnp.float32)
    zeros_c = jnp.zeros((4 * NP,), jnp.float32)

    def layer_w(i):
        ew1 = params[f"ew1_{i}"]
        wa = ew1[:, :D]
        wb = ew1[:, D:2 * D]
        wr = ew1[:, 2 * D:].reshape(1, D)
        b1 = params[f"eb1_{i}"].reshape(1, D)
        w2 = params[f"ew2_{i}"]
        b2 = params[f"eb2_{i}"].reshape(1, D)
        nw1 = params[f"nw1_{i}"]
        nw1a = nw1[:, :D]
        nw1b = nw1[:, D:]
        nb1 = params[f"nb1_{i}"].reshape(1, D)
        nw2 = params[f"nw2_{i}"]
        nb2 = params[f"nb2_{i}"].reshape(1, D)
        cw1 = params[f"cw1_{i}"]
        cb1 = params[f"cb1_{i}"].reshape(1, D)
        cw2 = params[f"cw2_{i}"]
        return wa, wb, wr, b1, w2, b2, nw1a, nw1b, nb1, nw2, nb2, cw1, cb1, cw2

    wa0, wb0 = layer_w(0)[:2]
    t1, t2 = _prep_stage(h_pad, ctab, wa0, wb0)

    for i in range(L):
        wa, wb, wr, b1, w2, b2, nw1a, nw1b, nb1, nw2, nb2, cw1, cb1, cw2 = layer_w(i)
        g1, g2 = _sc_gather(t1, t2, row, col)
        val_h, vct = _edge_stage(g1, g2, wr, b1, w2, b2, cw1, cb1, cw2)
        p0, p1, c0f, c1f = _sc_scatter(val_h, vct, row, zeros, zeros_c)
        c0 = c0f.reshape(4, NP)
        c1 = c1f.reshape(4, NP)
        if i < L - 1:
            wa_n, wb_n = layer_w(i + 1)[:2]
            h_pad, ctab, t1, t2 = _node_stage(
                p0, p1, c0, c1, h_pad, ctab, nw1a, nw1b, nb1, nw2, nb2,
                wa_n, wb_n)
        else:
            h_fin, c_fin = _final_stage(
                p0, p1, c0, c1, h_pad, ctab, nw1a, nw1b, nb1, nw2, nb2,
                params["out_w"], params["out_b"].reshape(1, D))

    return (h_fin[:N], c_fin[:N, :3])


# speedup vs baseline: 6.8686x; 1.1804x over previous
import jax
import jax.numpy as jnp
from jax import lax
from jax.experimental import pallas as pl
from jax.experimental.pallas import tpu as pltpu
from jax.experimental.pallas import tpu_sc as plsc

N = 10000
E = 320000
D = 128
L = 4
CP = 16
NP = 10240
NC = 2
NS = 16
NW = NC * NS
CH = 128
EH = E // 2


def _make_gather_body(ne):
    nchunk = ne // CH
    full = nchunk // NW
    extra = nchunk - full * NW
    npair = full // 2
    leftover = full - 2 * npair

    def gather_body(t1, t2, row, col, g1, g2,
                    irA, icA, irB, icB, b1A, b2A, b1B, b2B,
                    sIA, sIB, sGA, sGB, sWA, sWB):
        wid = lax.axis_index("s") * NC + lax.axis_index("c")

        def loads_i(m, s):
            irv, icv, b1, b2, sI, sG, sW = s
            base = (wid + m * NW) * CH
            pltpu.async_copy(row.at[pl.ds(base, CH)], irv, sI)
            pltpu.async_copy(col.at[pl.ds(base, CH)], icv, sI)

        def wait_i(s):
            irv, icv, b1, b2, sI, sG, sW = s
            pltpu.make_async_copy(row.at[pl.ds(0, CH)], irv, sI).wait()
            pltpu.make_async_copy(col.at[pl.ds(0, CH)], icv, sI).wait()

        def gathers(s):
            irv, icv, b1, b2, sI, sG, sW = s
            pltpu.async_copy(t1.at[irv], b1, sG)
            pltpu.async_copy(t2.at[icv], b2, sG)

        def wait_g(s):
            irv, icv, b1, b2, sI, sG, sW = s
            pltpu.make_async_copy(t1.at[irv], b1, sG).wait()
            pltpu.make_async_copy(t2.at[icv], b2, sG).wait()

        def wbs(m, s):
            irv, icv, b1, b2, sI, sG, sW = s
            base = (wid + m * NW) * CH
            pltpu.async_copy(b1, g1.at[pl.ds(base, CH)], sW)
            pltpu.async_copy(b2, g2.at[pl.ds(base, CH)], sW)

        def wait_w(s):
            irv, icv, b1, b2, sI, sG, sW = s
            pltpu.make_async_copy(b1, g1.at[pl.ds(0, CH)], sW).wait()
            pltpu.make_async_copy(b2, g2.at[pl.ds(0, CH)], sW).wait()

        setA = (irA, icA, b1A, b2A, sIA, sGA, sWA)
        setB = (irB, icB, b1B, b2B, sIB, sGB, sWB)

        def serial_chunk(ci):
            base = ci * CH
            pltpu.sync_copy(row.at[pl.ds(base, CH)], irA)
            pltpu.sync_copy(col.at[pl.ds(base, CH)], icA)
            cp1 = pltpu.async_copy(t1.at[irA], b1A, sGA)
            cp2 = pltpu.async_copy(t2.at[icA], b2A, sGB)
            cp1.wait()
            cp2.wait()
            pltpu.sync_copy(b1A, g1.at[pl.ds(base, CH)])
            pltpu.sync_copy(b2A, g2.at[pl.ds(base, CH)])

        loads_i(0, setA)

        def body(k, carry):
            wait_i(setA)

            @pl.when(k > 0)
            def _():
                wait_w(setA)

            gathers(setA)
            loads_i(2 * k + 1, setB)
            wait_g(setA)
            wbs(2 * k, setA)
            wait_i(setB)

            @pl.when(k > 0)
            def _():
                wait_w(setB)

            gathers(setB)

            @pl.when(k < npair - 1)
            def _():
                loads_i(2 * k + 2, setA)

            wait_g(setB)
            wbs(2 * k + 1, setB)
            return carry

        lax.fori_loop(0, npair, body, 0)
        wait_w(setA)
        wait_w(setB)

        if leftover:
            serial_chunk(wid + 2 * npair * NW)

        @pl.when(wid < extra)
        def _():
            serial_chunk(wid + full * NW)

    return gather_body


def _make_scatter_body(ne):
    nchunk_sc = (ne // CH) // NC
    full = nchunk_sc // NS
    extra = nchunk_sc - full * NS
    npair = full // 2
    leftover = full - 2 * npair

    def scatter_body(val, vct, row, zeros, zeros_c, out0, out1, cout0, cout1,
                     idxA, idxB, e1A, e2A, e3A, e1B, e2B, e3B,
                     bufA, bufB, tbA, tbB, acc, accc, sLA, sLB, sAA, sAB):
        cid = lax.axis_index("c")
        sid = lax.axis_index("s")
        rs = NP // NS
        cs = (4 * NP) // NS

        pltpu.sync_copy(zeros.at[pl.ds(sid * rs, rs)],
                        acc.at[pl.ds(sid * rs, rs)])
        pltpu.sync_copy(zeros_c.at[pl.ds(sid * cs, cs)],
                        accc.at[pl.ds(sid * cs, cs)])
        plsc.subcore_barrier()

        tbase = cid * nchunk_sc + sid
        setA = (idxA, e1A, e2A, e3A, bufA, tbA, sLA, sAA)
        setB = (idxB, e1B, e2B, e3B, bufB, tbB, sLB, sAB)

        def loads(m, s):
            idxv, e1, e2, e3, buf, tb, sL, sA = s
            c = tbase + m * NS
            base = c * CH
            pltpu.async_copy(row.at[pl.ds(base, CH)], idxv, sL)
            pltpu.async_copy(val.at[pl.ds(base, CH)], buf, sL)
            pltpu.async_copy(vct.at[c], tb, sL)

        def wait_loads(s):
            idxv, e1, e2, e3, buf, tb, sL, sA = s
            pltpu.make_async_copy(row.at[pl.ds(0, CH)], idxv, sL).wait()
            pltpu.make_async_copy(val.at[pl.ds(0, CH)], buf, sL).wait()
            pltpu.make_async_copy(vct.at[0], tb, sL).wait()

        def adds(s):
            idxv, e1, e2, e3, buf, tb, sL, sA = s
            for j in range(8):
                sl = pl.ds(j * 16, 16)
                v = idxv[sl]
                e1[sl] = v + NP
                e2[sl] = v + 2 * NP
                e3[sl] = v + 3 * NP
            pltpu.async_copy(buf, acc.at[idxv], sA, add=True)
            pltpu.async_copy(tb.at[0], accc.at[idxv], sA, add=True)
            pltpu.async_copy(tb.at[1], accc.at[e1], sA, add=True)
            pltpu.async_copy(tb.at[2], accc.at[e2], sA, add=True)
            pltpu.async_copy(tb.at[3], accc.at[e3], sA, add=True)

        def wait_adds(s):
            idxv, e1, e2, e3, buf, tb, sL, sA = s
            pltpu.make_async_copy(buf, acc.at[idxv], sA).wait()
            pltpu.make_async_copy(tb.at[0], accc.at[idxv], sA).wait()
            pltpu.make_async_copy(tb.at[1], accc.at[e1], sA).wait()
            pltpu.make_async_copy(tb.at[2], accc.at[e2], sA).wait()
            pltpu.make_async_copy(tb.at[3], accc.at[e3], sA).wait()

        def serial_chunk(c):
            base = c * CH
            pltpu.sync_copy(row.at[pl.ds(base, CH)], idxA)
            pltpu.sync_copy(val.at[pl.ds(base, CH)], bufA)
            pltpu.sync_copy(bufA, acc.at[idxA], add=True)
            pltpu.sync_copy(vct.at[c], tbA)
            for j in range(8):
                sl = pl.ds(j * 16, 16)
                v = idxA[sl]
                e1A[sl] = v + NP
                e2A[sl] = v + 2 * NP
                e3A[sl] = v + 3 * NP
            pltpu.sync_copy(tbA.at[0], accc.at[idxA], add=True)
            pltpu.sync_copy(tbA.at[1], accc.at[e1A], add=True)
            pltpu.sync_copy(tbA.at[2], accc.at[e2A], add=True)
            pltpu.sync_copy(tbA.at[3], accc.at[e3A], add=True)

        loads(0, setA)

        def body(k, carry):
            wait_loads(setA)

            @pl.when(k > 0)
            def _():
                wait_adds(setB)

            loads(2 * k + 1, setB)
            adds(setA)
            wait_loads(setB)
            wait_adds(setA)

            @pl.when(k < npair - 1)
            def _():
                loads(2 * k + 2, setA)

            adds(setB)
            return carry

        lax.fori_loop(0, npair, body, 0)
        wait_adds(setB)

        if leftover:
            serial_chunk(tbase + 2 * npair * NS)

        @pl.when(sid < extra)
        def _():
            serial_chunk(tbase + full * NS)

        plsc.subcore_barrier()

        @pl.when(cid == 0)
        def _():
            pltpu.sync_copy(acc.at[pl.ds(sid * rs, rs)],
                            out0.at[pl.ds(sid * rs, rs)])
            pltpu.sync_copy(accc.at[pl.ds(sid * cs, cs)],
                            cout0.at[pl.ds(sid * cs, cs)])

        @pl.when(cid == 1)
        def _():
            pltpu.sync_copy(acc.at[pl.ds(sid * rs, rs)],
                            out1.at[pl.ds(sid * rs, rs)])
            pltpu.sync_copy(accc.at[pl.ds(sid * cs, cs)],
                            cout1.at[pl.ds(sid * cs, cs)])

    return scatter_body


_sc_cache = {}


def _sc_gather(t1, t2, row, col):
    ne = row.shape[0]
    key = ("gather", ne)
    if key not in _sc_cache:
        mesh = plsc.VectorSubcoreMesh(core_axis_name="c", subcore_axis_name="s")
        _sc_cache[key] = pl.kernel(
            _make_gather_body(ne),
            mesh=mesh,
            out_type=(
                jax.ShapeDtypeStruct((ne, D), jnp.int32),
                jax.ShapeDtypeStruct((ne, D), jnp.int32),
            ),
            scratch_types=(
                [pltpu.VMEM((CH,), jnp.int32) for _ in range(4)]
                + [pltpu.VMEM((CH, D), jnp.int32) for _ in range(4)]
                + [pltpu.SemaphoreType.DMA for _ in range(6)]
            ),
        )
    return _sc_cache[key](t1, t2, row, col)


def _sc_scatter(val, vct, row, zeros, zeros_c):
    ne = row.shape[0]
    key = ("scatter", ne)
    if key not in _sc_cache:
        mesh = plsc.VectorSubcoreMesh(core_axis_name="c", subcore_axis_name="s")
        _sc_cache[key] = pl.kernel(
            _make_scatter_body(ne),
            mesh=mesh,
            out_type=(
                jax.ShapeDtypeStruct((NP, D), jnp.float32),
                jax.ShapeDtypeStruct((NP, D), jnp.float32),
                jax.ShapeDtypeStruct((4 * NP,), jnp.float32),
                jax.ShapeDtypeStruct((4 * NP,), jnp.float32),
            ),
            scratch_types=(
                [pltpu.VMEM((CH,), jnp.int32) for _ in range(8)]
                + [pltpu.VMEM((CH, D), jnp.float32) for _ in range(2)]
                + [pltpu.VMEM((8, CH), jnp.float32) for _ in range(2)]
                + [pltpu.VMEM_SHARED((NP, D), jnp.float32),
                   pltpu.VMEM_SHARED((4 * NP,), jnp.float32)]
                + [pltpu.SemaphoreType.DMA for _ in range(4)]
            ),
        )
    return _sc_cache[key](val, vct, row, zeros, zeros_c)


def _pack_words(payload, coord3):
    pay = lax.convert_element_type(
        lax.bitcast_convert_type(payload.astype(jnp.bfloat16), jnp.uint16),
        jnp.uint32)
    cbits = lax.bitcast_convert_type(coord3, jnp.uint32)
    hi = cbits >> 16
    lo = cbits & 0xFFFF
    b = payload.shape[0]
    z13 = jnp.zeros((b, 13), jnp.uint32)
    z109 = jnp.zeros((b, 109), jnp.uint32)
    cplane = jnp.concatenate([hi, z13, lo, z109], axis=1)
    return lax.bitcast_convert_type(pay | (cplane << 16), jnp.int32)


def _unpack_words(words):
    w = lax.bitcast_convert_type(words, jnp.uint32)
    pay = lax.bitcast_convert_type(
        lax.convert_element_type(w & 0xFFFF, jnp.uint16),
        jnp.bfloat16).astype(jnp.float32)
    cplane = w >> 16
    coord = lax.bitcast_convert_type(
        (cplane[:, 0:3] << 16) | cplane[:, 16:19], jnp.float32)
    return pay, coord


BE = 3200


def _sigmoid(x):
    return 1.0 / (1.0 + jnp.exp(-x))


def _silu(x):
    return x * _sigmoid(x)


def _mmt(x, w):
    return lax.dot_general(x, w, (((1,), (1,)), ((), ())),
                           preferred_element_type=jnp.float32)


def _edge_body(g1_ref, g2_ref, wr_ref, b1_ref, w2_ref, b2_ref,
               cw1_ref, cb1_ref, cw2_ref, vh_ref, vct_ref):
    h1, c1 = _unpack_words(g1_ref[...])
    h2, c2 = _unpack_words(g2_ref[...])
    hs = h1 + h2
    cd = c1 - c2
    radial = jnp.sum(cd * cd, axis=1, keepdims=True)
    t = _silu(hs + radial * wr_ref[...] + b1_ref[...])
    ef = _silu(_mmt(t, w2_ref[...]) + b2_ref[...])
    cm = _silu(_mmt(ef, cw1_ref[...]) + cb1_ref[...])
    cms = _mmt(cm, cw2_ref[...])
    vh_ref[...] = ef
    t8 = jnp.concatenate([cd * cms, jnp.ones((BE, 1), jnp.float32),
                          jnp.zeros((BE, 4), jnp.float32)], axis=1)
    vct_ref[...] = jnp.swapaxes(t8.reshape(BE // CH, CH, 8), 1, 2)


def _edge_stage(g1, g2, wr, b1, w2, b2, cw1, cb1, cw2):
    ne = g1.shape[0]
    wspec = lambda shape: pl.BlockSpec(shape, lambda i: (0, 0))
    return pl.pallas_call(
        _edge_body,
        grid=(ne // BE,),
        in_specs=[
            pl.BlockSpec((BE, D), lambda i: (i, 0)),
            pl.BlockSpec((BE, D), lambda i: (i, 0)),
            wspec((1, D)), wspec((1, D)), wspec((D, D)), wspec((1, D)),
            wspec((D, D)), wspec((1, D)), wspec((1, D)),
        ],
        out_specs=[
            pl.BlockSpec((BE, D), lambda i: (i, 0)),
            pl.BlockSpec((BE // CH, 8, CH), lambda i: (i, 0, 0)),
        ],
        out_shape=[
            jax.ShapeDtypeStruct((ne, D), jnp.float32),
            jax.ShapeDtypeStruct((ne // CH, 8, CH), jnp.float32),
        ],
    )(g1, g2, wr, b1, w2, b2, cw1, cb1, cw2)


BN = 1024


def _node_common(p0, p1, p2, p3, c0, c1, c2, c3, ct, h,
                 nw1a, nw1b, nb1, nw2, nb2):
    agg = (p0 + p1) + (p2 + p3)
    sc = jnp.swapaxes((c0 + c1) + (c2 + c3), 0, 1)
    cnt = jnp.maximum(sc[:, 3:4], 1.0)
    cold = jnp.swapaxes(ct, 0, 1)[:, :3]
    coord_new = cold + sc[:, :3] / cnt
    m = _silu(_mmt(h, nw1a) + _mmt(agg, nw1b) + nb1)
    h_new = h + _mmt(m, nw2) + nb2
    return h_new, coord_new


def _ctab_block(coord_new):
    b = coord_new.shape[0]
    return jnp.swapaxes(jnp.concatenate(
        [coord_new, jnp.zeros((b, 1), jnp.float32)], axis=1), 0, 1)


def _node_body(p0_ref, p1_ref, p2_ref, p3_ref,
               c0_ref, c1_ref, c2_ref, c3_ref, h_ref, ct_ref,
               nw1a_ref, nw1b_ref, nb1_ref, nw2_ref, nb2_ref,
               wa_ref, wb_ref, h_out, ct_out, t1_out, t2_out):
    h_new, coord_new = _node_common(
        p0_ref[...], p1_ref[...], p2_ref[...], p3_ref[...],
        c0_ref[...], c1_ref[...], c2_ref[...], c3_ref[...], ct_ref[...],
        h_ref[...], nw1a_ref[...], nw1b_ref[...], nb1_ref[...],
        nw2_ref[...], nb2_ref[...])
    h_out[...] = h_new
    ct_out[...] = _ctab_block(coord_new)
    t1_out[...] = _pack_words(_mmt(h_new, wa_ref[...]), coord_new)
    t2_out[...] = _pack_words(_mmt(h_new, wb_ref[...]), coord_new)


def _node_stage(ps, cs, h, ctab, nw1a, nw1b, nb1, nw2, nb2, wa, wb):
    wspec = lambda shape: pl.BlockSpec(shape, lambda i: (0, 0))
    bspec = lambda w: pl.BlockSpec((BN, w), lambda i: (i, 0))
    cspec = pl.BlockSpec((4, BN), lambda i: (0, i))
    return pl.pallas_call(
        _node_body,
        grid=(NP // BN,),
        in_specs=[
            bspec(D), bspec(D), bspec(D), bspec(D),
            cspec, cspec, cspec, cspec, bspec(D), cspec,
            wspec((D, D)), wspec((D, D)), wspec((1, D)),
            wspec((D, D)), wspec((1, D)), wspec((D, D)), wspec((D, D)),
        ],
        out_specs=[
            bspec(D), cspec, bspec(D), bspec(D),
        ],
        out_shape=[
            jax.ShapeDtypeStruct((NP, D), jnp.float32),
            jax.ShapeDtypeStruct((4, NP), jnp.float32),
            jax.ShapeDtypeStruct((NP, D), jnp.int32),
            jax.ShapeDtypeStruct((NP, D), jnp.int32),
        ],
    )(*ps, *cs, h, ctab, nw1a, nw1b, nb1, nw2, nb2, wa, wb)


def _final_body(p0_ref, p1_ref, p2_ref, p3_ref,
                c0_ref, c1_ref, c2_ref, c3_ref, h_ref, ct_ref,
                nw1a_ref, nw1b_ref, nb1_ref, nw2_ref, nb2_ref,
                ow_ref, ob_ref, h_out, c_out):
    h_new, coord_new = _node_common(
        p0_ref[...], p1_ref[...], p2_ref[...], p3_ref[...],
        c0_ref[...], c1_ref[...], c2_ref[...], c3_ref[...], ct_ref[...],
        h_ref[...], nw1a_ref[...], nw1b_ref[...], nb1_ref[...],
        nw2_ref[...], nb2_ref[...])
    h_out[...] = _mmt(h_new, ow_ref[...]) + ob_ref[...]
    c_out[...] = jnp.concatenate(
        [coord_new, jnp.zeros((BN, CP - 3), jnp.float32)], axis=1)


def _final_stage(ps, cs, h, ctab, nw1a, nw1b, nb1, nw2, nb2, ow, ob):
    wspec = lambda shape: pl.BlockSpec(shape, lambda i: (0, 0))
    bspec = lambda w: pl.BlockSpec((BN, w), lambda i: (i, 0))
    cspec = pl.BlockSpec((4, BN), lambda i: (0, i))
    return pl.pallas_call(
        _final_body,
        grid=(NP // BN,),
        in_specs=[
            bspec(D), bspec(D), bspec(D), bspec(D),
            cspec, cspec, cspec, cspec, bspec(D), cspec,
            wspec((D, D)), wspec((D, D)), wspec((1, D)),
            wspec((D, D)), wspec((1, D)), wspec((D, D)), wspec((1, D)),
        ],
        out_specs=[bspec(D), bspec(CP)],
        out_shape=[
            jax.ShapeDtypeStruct((NP, D), jnp.float32),
            jax.ShapeDtypeStruct((NP, CP), jnp.float32),
        ],
    )(*ps, *cs, h, ctab, nw1a, nw1b, nb1, nw2, nb2, ow, ob)


def _prep_body(h_ref, ct_ref, wa_ref, wb_ref, t1_out, t2_out):
    coord3 = jnp.swapaxes(ct_ref[...], 0, 1)[:, :3]
    t1_out[...] = _pack_words(_mmt(h_ref[...], wa_ref[...]), coord3)
    t2_out[...] = _pack_words(_mmt(h_ref[...], wb_ref[...]), coord3)


def _prep_stage(h, ctab, wa, wb):
    wspec = lambda shape: pl.BlockSpec(shape, lambda i: (0, 0))
    return pl.pallas_call(
        _prep_body,
        grid=(NP // BN,),
        in_specs=[
            pl.BlockSpec((BN, D), lambda i: (i, 0)),
            pl.BlockSpec((4, BN), lambda i: (0, i)),
            wspec((D, D)), wspec((D, D)),
        ],
        out_specs=[
            pl.BlockSpec((BN, D), lambda i: (i, 0)),
            pl.BlockSpec((BN, D), lambda i: (i, 0)),
        ],
        out_shape=[
            jax.ShapeDtypeStruct((NP, D), jnp.int32),
            jax.ShapeDtypeStruct((NP, D), jnp.int32),
        ],
    )(h, ctab, wa, wb)


def kernel(h, x, edges, params):
    row = edges[0]
    col = edges[1]
    row0, col0 = row[:EH], col[:EH]
    row1, col1 = row[EH:], col[EH:]
    h_pad = jnp.pad(h, ((0, NP - N), (0, 0)))
    ctab = jnp.pad(x, ((0, NP - N), (0, 1))).T
    zeros = jnp.zeros((NP, D), jnp.float32)
    zeros_c = jnp.zeros((4 * NP,), jnp.float32)

    def layer_w(i):
        ew1 = params[f"ew1_{i}"]
        wa = ew1[:, :D]
        wb = ew1[:, D:2 * D]
        wr = ew1[:, 2 * D:].reshape(1, D)
        b1 = params[f"eb1_{i}"].reshape(1, D)
        w2 = params[f"ew2_{i}"]
        b2 = params[f"eb2_{i}"].reshape(1, D)
        nw1 = params[f"nw1_{i}"]
        nw1a = nw1[:, :D]
        nw1b = nw1[:, D:]
        nb1 = params[f"nb1_{i}"].reshape(1, D)
        nw2 = params[f"nw2_{i}"]
        nb2 = params[f"nb2_{i}"].reshape(1, D)
        cw1 = params[f"cw1_{i}"]
        cb1 = params[f"cb1_{i}"].reshape(1, D)
        cw2 = params[f"cw2_{i}"]
        return wa, wb, wr, b1, w2, b2, nw1a, nw1b, nb1, nw2, nb2, cw1, cb1, cw2

    wa0, wb0 = layer_w(0)[:2]
    t1, t2 = _prep_stage(h_pad, ctab, wa0, wb0)

    for i in range(L):
        wa, wb, wr, b1, w2, b2, nw1a, nw1b, nb1, nw2, nb2, cw1, cb1, cw2 = layer_w(i)
        g1a, g2a = _sc_gather(t1, t2, row0, col0)
        vha, vcta = _edge_stage(g1a, g2a, wr, b1, w2, b2, cw1, cb1, cw2)
        g1b, g2b = _sc_gather(t1, t2, row1, col1)
        p0a, p1a, c0a, c1a = _sc_scatter(vha, vcta, row0, zeros, zeros_c)
        vhb, vctb = _edge_stage(g1b, g2b, wr, b1, w2, b2, cw1, cb1, cw2)
        p0b, p1b, c0b, c1b = _sc_scatter(vhb, vctb, row1, zeros, zeros_c)
        ps = (p0a, p1a, p0b, p1b)
        cs = tuple(c.reshape(4, NP) for c in (c0a, c1a, c0b, c1b))
        if i < L - 1:
            wa_n, wb_n = layer_w(i + 1)[:2]
            h_pad, ctab, t1, t2 = _node_stage(
                ps, cs, h_pad, ctab, nw1a, nw1b, nb1, nw2, nb2,
                wa_n, wb_n)
        else:
            h_fin, c_fin = _final_stage(
                ps, cs, h_pad, ctab, nw1a, nw1b, nb1, nw2, nb2,
                params["out_w"], params["out_b"].reshape(1, D))

    return (h_fin[:N], c_fin[:N, :3])
